# Initial kernel scaffold; baseline (speedup 1.0000x reference)
import jax
import jax.numpy as jnp
from jax.experimental import pallas as pl


def kernel(x, W1, a_src1, a_dst1, b1, W2, a_src2, a_dst2, b2, Wc1, bc1, Wc2, bc2, edge_index, batch_vec):
    raise NotImplementedError("write your pallas kernel here")


# speedup vs baseline: 9.7943x; 9.7943x over previous
import functools
import jax
import jax.numpy as jnp
from jax import lax
from jax.experimental import pallas as pl
from jax.experimental.pallas import tpu as pltpu
from jax.experimental.pallas import tpu_sc as plsc

N = 10000
E = 160000
N_GRAPHS = 200
IN_CH = 256
HID = 128
HEADS = 8
OUT_CH = 256
N_CLASSES = 14

NC = 2
NS = 16
CK = 80
EPT = E // NS
NCHUNK = EPT // CK
NPT = 640
FC = 80
NPAD = NS * NPT


def _tile_rows(sidx):
    nstart = sidx * NPT
    ncht = jnp.where(sidx < NS - 1, NPT // FC, (N - NPT * (NS - 1)) // FC)
    return nstart, ncht


def _mm_aux_body(x_ref, w_ref, a_ref, h_ref, aux_ref):
    h = jnp.dot(x_ref[...], w_ref[...], preferred_element_type=jnp.float32)
    h_ref[...] = h
    aux_ref[...] = jnp.dot(h, a_ref[...], preferred_element_type=jnp.float32)


def _mm_aux(x, wt, amat, bn):
    n, k = x.shape
    m = wt.shape[1]
    aw = amat.shape[1]
    return pl.pallas_call(
        _mm_aux_body,
        grid=(n // bn,),
        in_specs=[
            pl.BlockSpec((bn, k), lambda i: (i, 0)),
            pl.BlockSpec((k, m), lambda i: (0, 0)),
            pl.BlockSpec((m, aw), lambda i: (0, 0)),
        ],
        out_specs=[
            pl.BlockSpec((bn, m), lambda i: (i, 0)),
            pl.BlockSpec((bn, aw), lambda i: (i, 0)),
        ],
        out_shape=[
            jax.ShapeDtypeStruct((n, m), jnp.float32),
            jax.ShapeDtypeStruct((n, aw), jnp.float32),
        ],
    )(x, wt, amat)


def _pool_body(h2_ref, bv_ref, wc1_ref, bc1_ref, wc2_ref, bc2_ref,
               out_ref, sum_acc, cnt_acc):
    i = pl.program_id(0)

    @pl.when(i == 0)
    def _():
        sum_acc[...] = jnp.zeros_like(sum_acc)
        cnt_acc[...] = jnp.zeros_like(cnt_acc)

    bv = bv_ref[...]
    oh = (bv == lax.broadcasted_iota(jnp.int32, (1, N_GRAPHS), 1)
          ).astype(jnp.float32)
    h2 = h2_ref[...]
    dn = (((0,), (0,)), ((), ()))
    sum_acc[...] += lax.dot_general(oh, h2, dn,
                                    preferred_element_type=jnp.float32)
    cnt_acc[...] += lax.dot_general(oh, jnp.ones_like(h2), dn,
                                    preferred_element_type=jnp.float32)

    @pl.when(i == pl.num_programs(0) - 1)
    def _():
        pooled = sum_acc[...] / jnp.maximum(cnt_acc[...], 1.0)
        z = jnp.maximum(
            jnp.dot(pooled, wc1_ref[...], preferred_element_type=jnp.float32)
            + bc1_ref[...], 0.0)
        out_ref[...] = jnp.dot(
            z, wc2_ref[...], preferred_element_type=jnp.float32) + bc2_ref[...]


def _pool_classify(h2, bv, wc1t, bc1, wc2tp, bc2p, bn):
    n, d = h2.shape
    return pl.pallas_call(
        _pool_body,
        grid=(n // bn,),
        in_specs=[
            pl.BlockSpec((bn, d), lambda i: (i, 0)),
            pl.BlockSpec((bn, 1), lambda i: (i, 0)),
            pl.BlockSpec(wc1t.shape, lambda i: (0, 0)),
            pl.BlockSpec(bc1.shape, lambda i: (0, 0)),
            pl.BlockSpec(wc2tp.shape, lambda i: (0, 0)),
            pl.BlockSpec(bc2p.shape, lambda i: (0, 0)),
        ],
        out_specs=pl.BlockSpec((N_GRAPHS, 128), lambda i: (0, 0)),
        out_shape=jax.ShapeDtypeStruct((N_GRAPHS, 128), jnp.float32),
        scratch_shapes=[
            pltpu.VMEM((N_GRAPHS, d), jnp.float32),
            pltpu.VMEM((N_GRAPHS, d), jnp.float32),
        ],
    )(h2, bv, wc1t, bc1, wc2tp, bc2p)


def _den_pass(aux128, srcm, dstm, hpd, shared_aux):
    mesh = plsc.VectorSubcoreMesh(core_axis_name="c", subcore_axis_name="s",
                                  num_cores=NC, num_subcores=NS)

    @functools.partial(
        pl.kernel, mesh=mesh,
        compiler_params=pltpu.CompilerParams(needs_layout_passes=False),
        out_type=jax.ShapeDtypeStruct((N, NC * 128), jnp.float32),
        scratch_types=[
            pltpu.VMEM_SHARED((N, 128), jnp.float32),
            pltpu.VMEM((CK,), jnp.int32),
            pltpu.VMEM((CK,), jnp.int32),
            pltpu.VMEM((CK, 128), jnp.float32),
            pltpu.VMEM((CK, 128), jnp.float32),
            pltpu.VMEM((CK, 128), jnp.float32),
            pltpu.SemaphoreType.DMA,
        ],
    )
    def k(aux_h, srcm_h, dstm_h, out_h,
          acc, src_b, dst_b, gs, gd, stage, sem):
        cidx = lax.axis_index("c")
        sidx = lax.axis_index("s")
        zero16 = jnp.zeros((16,), jnp.float32)
        zero16i = jnp.zeros((16,), jnp.int32)
        iota16 = lax.iota(jnp.int32, 16)
        nstart, ncht = _tile_rows(sidx)

        def zrow(r, _):
            for j in range(8):
                gs[r, pl.ds(j * 16, 16)] = zero16
                stage[r, pl.ds(j * 16, 16)] = zero16
            return 0
        lax.fori_loop(0, CK, zrow, 0)

        def zchunk(i, _):
            r0 = pl.multiple_of(nstart + i * FC, 8)
            pltpu.sync_copy(gs, acc.at[pl.ds(r0, FC)])
            return 0
        lax.fori_loop(0, ncht, zchunk, 0)
        plsc.subcore_barrier()

        def echunk(ck, _):
            e0 = pl.multiple_of(sidx * EPT + ck * CK, 8)
            pltpu.sync_copy(srcm_h.at[pl.ds(e0, CK)], src_b)
            pltpu.sync_copy(dstm_h.at[pl.ds(e0, CK)], dst_b)
            cps = pltpu.async_copy(aux_h.at[src_b], gs, sem)
            cpd = pltpu.async_copy(aux_h.at[dst_b], gd, sem)
            cps.wait()
            cpd.wait()
            for g in range(CK // 16):
                rows = iota16 + g * 16
                for h in range(hpd):
                    if shared_aux:
                        cs, cd = zero16i, zero16i + 1
                    else:
                        cs = zero16i + (cidx * hpd + h)
                        cd = zero16i + (NC * hpd + cidx * hpd + h)
                    a = (plsc.load_gather(gs, [rows, cs])
                         + plsc.load_gather(gd, [rows, cd]))
                    a = jnp.where(a > 0, a, 0.2 * a)
                    h16 = jnp.full((16,), h, jnp.int32)
                    plsc.store_scatter(stage, [rows, h16], jnp.exp(a))
            pltpu.sync_copy(stage, acc.at[dst_b], add=True)
            return 0
        lax.fori_loop(0, NCHUNK, echunk, 0)
        plsc.subcore_barrier()

        def wchunk(i, _):
            n0 = pl.multiple_of(nstart + i * FC, 8)
            c0 = pl.multiple_of(cidx * 128, 128)
            pltpu.sync_copy(acc.at[pl.ds(n0, FC)], gd)
            pltpu.sync_copy(gd, out_h.at[pl.ds(n0, FC), pl.ds(c0, 128)])
            return 0
        lax.fori_loop(0, ncht, wchunk, 0)

    return k(aux128, srcm, dstm)


def _edge_gat(hv, auxt3, srcm, dstm, bias, den, nseg, hpc, shared_aux):
    ndc = den.shape[0]
    mesh = plsc.VectorSubcoreMesh(core_axis_name="c", subcore_axis_name="s",
                                  num_cores=NC, num_subcores=NS)

    @functools.partial(
        pl.kernel, mesh=mesh,
        compiler_params=pltpu.CompilerParams(needs_layout_passes=False),
        out_type=jax.ShapeDtypeStruct((N, nseg * 128), jnp.float32),
        scratch_types=[
            pltpu.VMEM_SHARED((N, 128), jnp.float32),
            pltpu.VMEM((N,), jnp.float32),
            pltpu.VMEM((N,), jnp.float32),
            pltpu.VMEM((NPT,), jnp.float32),
            pltpu.VMEM((128,), jnp.float32),
            pltpu.VMEM((CK,), jnp.int32),
            pltpu.VMEM((CK,), jnp.int32),
            pltpu.VMEM((CK,), jnp.int32),
            pltpu.VMEM((CK,), jnp.float32),
            pltpu.VMEM((CK, 128), jnp.float32),
            pltpu.VMEM((FC, 128), jnp.float32),
            pltpu.VMEM((FC,), jnp.float32),
            pltpu.VMEM((FC,), jnp.float32),
            pltpu.SemaphoreType.DMA,
        ],
    )
    def k(hv_h, auxt_h, srcm_h, dstm_h, bias_h, den_h, out_h,
          acc, asrc_v, adst_v, den_v, bias_v, src_b, dst_b, idx_b, wbuf,
          gbuf, hbuf, wsbuf, invbuf, gsem):
        cidx = lax.axis_index("c")
        sidx = lax.axis_index("s")
        zero16 = jnp.zeros((16,), jnp.float32)
        zero16i = jnp.zeros((16,), jnp.int32)
        iota16 = lax.iota(jnp.int32, 16)
        nstart, ncht = _tile_rows(sidx)

        def head_body(hd, _):
            hdg = cidx * hpc + hd
            if shared_aux:
                a_row = jnp.int32(0)
                b_row = jnp.int32(1)
                d_row = jnp.int32(0)
            else:
                a_row = hdg
                b_row = nseg + hdg
                d_row = hdg
            pltpu.sync_copy(
                den_h.at[d_row, 0,
                         pl.ds(pl.multiple_of(nstart, 8), NPT)], den_v)
            pltpu.sync_copy(auxt_h.at[a_row, 0], asrc_v)
            pltpu.sync_copy(auxt_h.at[b_row, 0], adst_v)
            pltpu.sync_copy(bias_h.at[hdg, 0], bias_v)
            bj = [bias_v[pl.ds(j * 16, 16)] for j in range(8)]

            def zrow(r, _):
                for j in range(8):
                    hbuf[r, pl.ds(j * 16, 16)] = zero16
                return 0
            lax.fori_loop(0, FC, zrow, 0)

            def zchunk(i, _):
                r0 = pl.multiple_of(nstart + i * FC, 8)
                pltpu.sync_copy(hbuf, acc.at[pl.ds(r0, FC)])
                return 0
            lax.fori_loop(0, ncht, zchunk, 0)
            plsc.subcore_barrier()

            def echunk(ck, _):
                e0 = pl.multiple_of(sidx * EPT + ck * CK, 8)
                pltpu.sync_copy(srcm_h.at[pl.ds(e0, CK)], src_b)
                pltpu.sync_copy(dstm_h.at[pl.ds(e0, CK)], dst_b)
                for g in range(CK // 16):
                    s16 = src_b[pl.ds(g * 16, 16)]
                    idx_b[pl.ds(g * 16, 16)] = s16 * nseg + hdg
                cp = pltpu.async_copy(hv_h.at[idx_b], gbuf, gsem)
                for g in range(CK // 16):
                    s16 = src_b[pl.ds(g * 16, 16)]
                    d16 = dst_b[pl.ds(g * 16, 16)]
                    a = (plsc.load_gather(asrc_v, [s16])
                         + plsc.load_gather(adst_v, [d16]))
                    a = jnp.where(a > 0, a, 0.2 * a)
                    wbuf[pl.ds(g * 16, 16)] = jnp.exp(a)
                cp.wait()

                def scale(g, _):
                    w16 = wbuf[pl.ds(g * 16, 16)]
                    for e in range(16):
                        w_e = w16[e]
                        row = g * 16 + e
                        for j in range(8):
                            sl = pl.ds(j * 16, 16)
                            gbuf[row, sl] = w_e * gbuf[row, sl]
                    return 0
                lax.fori_loop(0, CK // 16, scale, 0)
                pltpu.sync_copy(gbuf, acc.at[dst_b], add=True)
                return 0
            lax.fori_loop(0, NCHUNK, echunk, 0)
            plsc.subcore_barrier()

            def fchunk(i, _):
                n0 = pl.multiple_of(nstart + i * FC, 8)
                pltpu.sync_copy(acc.at[pl.ds(n0, FC)], gbuf)
                for g in range(FC // 16):
                    i16 = iota16 + (n0 + g * 16)
                    idx_b[pl.ds(g * 16, 16)] = i16 * nseg + hdg
                cpf = pltpu.async_copy(hv_h.at[idx_b], hbuf, gsem)
                for g in range(FC // 16):
                    sl = pl.ds(g * 16, 16)
                    o16 = pl.multiple_of(n0 + g * 16, 8)
                    a = asrc_v[pl.ds(o16, 16)] + adst_v[pl.ds(o16, 16)]
                    a = jnp.where(a > 0, a, 0.2 * a)
                    ws = jnp.exp(a)
                    den16 = den_v[pl.ds(
                        pl.multiple_of(i * FC + g * 16, 8), 16)]
                    wsbuf[sl] = ws
                    invbuf[sl] = 1.0 / (den16 + ws + 1e-16)
                cpf.wait()

                def fnode(g, _):
                    ws16 = wsbuf[pl.ds(g * 16, 16)]
                    inv16 = invbuf[pl.ds(g * 16, 16)]
                    for e in range(16):
                        ws = ws16[e]
                        inv = inv16[e]
                        row = g * 16 + e
                        for j in range(8):
                            sl = pl.ds(j * 16, 16)
                            v = ((gbuf[row, sl] + ws * hbuf[row, sl]) * inv
                                 + bj[j])
                            hbuf[row, sl] = jnp.where(
                                v > 0, v, jnp.exp(v) - 1.0)
                    return 0
                lax.fori_loop(0, FC // 16, fnode, 0)
                c0 = pl.multiple_of(hdg * 128, 128)
                pltpu.sync_copy(hbuf, out_h.at[pl.ds(n0, FC), pl.ds(c0, 128)])
                return 0
            lax.fori_loop(0, ncht, fchunk, 0)
            plsc.subcore_barrier()
            return 0

        lax.fori_loop(0, hpc, head_body, 0)

    return k(hv, auxt3, srcm, dstm, bias, den)


def kernel(x, W1, a_src1, a_dst1, b1, W2, a_src2, a_dst2, b2,
           Wc1, bc1, Wc2, bc2, edge_index, batch_vec):
    src = edge_index[0].astype(jnp.int32)
    dst = edge_index[1].astype(jnp.int32)

    eye1 = jnp.eye(HEADS, dtype=jnp.float32)
    bd_src1 = (a_src1[0][:, :, None] * eye1[:, None, :]).reshape(
        HEADS * HID, HEADS)
    bd_dst1 = (a_dst1[0][:, :, None] * eye1[:, None, :]).reshape(
        HEADS * HID, HEADS)
    amat1 = jnp.concatenate(
        [bd_src1, bd_dst1,
         jnp.zeros((HEADS * HID, 112), jnp.float32)], axis=1)
    amat2 = jnp.concatenate(
        [a_src2[0].T, a_dst2[0].T,
         jnp.zeros((OUT_CH, 126), jnp.float32)], axis=1)

    pad = ((0, NPAD - N), (0, 0))

    H1, AUX1 = _mm_aux(x, W1.T, amat1, bn=1000)
    denp1 = _den_pass(AUX1, src, dst, hpd=HEADS // NC, shared_aux=False)
    den1 = jnp.pad(jnp.concatenate(
        [denp1[:, 0:4], denp1[:, 128:132]], axis=1),
        pad).T.reshape(HEADS, 1, NPAD)
    auxt31 = AUX1[:, :2 * HEADS].T.reshape(2 * HEADS, 1, N)
    h1 = _edge_gat(H1.reshape(N * HEADS, HID), auxt31, src, dst,
                   b1.reshape(HEADS, 1, HID), den1,
                   nseg=HEADS, hpc=HEADS // NC, shared_aux=False)

    H2, AUX2 = _mm_aux(h1, W2.T, amat2, bn=1000)
    denp2 = _den_pass(AUX2, src, dst, hpd=1, shared_aux=True)
    den2 = jnp.pad(denp2[:, 0:1], pad).T.reshape(1, 1, NPAD)
    auxt32 = AUX2[:, :2].T.reshape(2, 1, N)
    h2 = _edge_gat(H2.reshape(N * 2, 128), auxt32, src, dst,
                   b2.reshape(2, 1, 128), den2,
                   nseg=2, hpc=1, shared_aux=True)

    bv = batch_vec.astype(jnp.int32).reshape(N, 1)
    wc2tp = jnp.concatenate(
        [Wc2.T, jnp.zeros((OUT_CH, 128 - N_CLASSES), jnp.float32)], axis=1)
    bc2p = jnp.concatenate(
        [bc2, jnp.zeros((128 - N_CLASSES,), jnp.float32)]).reshape(1, 128)
    logits_p = _pool_classify(h2, bv, Wc1.T, bc1.reshape(1, OUT_CH),
                              wc2tp, bc2p, bn=1000)
    return logits_p[:, :N_CLASSES]


# speedup vs baseline: 14.6219x; 1.4929x over previous
import functools
import jax
import jax.numpy as jnp
from jax import lax
from jax.experimental import pallas as pl
from jax.experimental.pallas import tpu as pltpu
from jax.experimental.pallas import tpu_sc as plsc

N = 10000
E = 160000
N_GRAPHS = 200
IN_CH = 256
HID = 128
HEADS = 8
OUT_CH = 256
N_CLASSES = 14

NC = 2
NS = 16
CK = 80
EPT = E // NS
NCHUNK = EPT // CK
NPAIR = (NCHUNK - 1) // 2
NPT = 640
FC = 80
NPAD = NS * NPT


def _tile_rows(sidx):
    nstart = sidx * NPT
    ncht = jnp.where(sidx < NS - 1, NPT // FC, (N - NPT * (NS - 1)) // FC)
    return nstart, ncht


def _mm_aux_body(x_ref, w_ref, a_ref, h_ref, aux_ref):
    h = jnp.dot(x_ref[...], w_ref[...], preferred_element_type=jnp.float32)
    h_ref[...] = h
    aux_ref[...] = jnp.dot(h, a_ref[...], preferred_element_type=jnp.float32)


def _mm_aux(x, wt, amat, bn):
    n, k = x.shape
    m = wt.shape[1]
    aw = amat.shape[1]
    return pl.pallas_call(
        _mm_aux_body,
        grid=(n // bn,),
        in_specs=[
            pl.BlockSpec((bn, k), lambda i: (i, 0)),
            pl.BlockSpec((k, m), lambda i: (0, 0)),
            pl.BlockSpec((m, aw), lambda i: (0, 0)),
        ],
        out_specs=[
            pl.BlockSpec((bn, m), lambda i: (i, 0)),
            pl.BlockSpec((bn, aw), lambda i: (i, 0)),
        ],
        out_shape=[
            jax.ShapeDtypeStruct((n, m), jnp.float32),
            jax.ShapeDtypeStruct((n, aw), jnp.float32),
        ],
    )(x, wt, amat)


def _pool_body(h2_ref, bv_ref, wc1_ref, bc1_ref, wc2_ref, bc2_ref,
               out_ref, sum_acc, cnt_acc):
    i = pl.program_id(0)

    @pl.when(i == 0)
    def _():
        sum_acc[...] = jnp.zeros_like(sum_acc)
        cnt_acc[...] = jnp.zeros_like(cnt_acc)

    bv = bv_ref[...]
    oh = (bv == lax.broadcasted_iota(jnp.int32, (1, N_GRAPHS), 1)
          ).astype(jnp.float32)
    h2 = h2_ref[...]
    dn = (((0,), (0,)), ((), ()))
    sum_acc[...] += lax.dot_general(oh, h2, dn,
                                    preferred_element_type=jnp.float32)
    cnt_acc[...] += lax.dot_general(oh, jnp.ones_like(h2), dn,
                                    preferred_element_type=jnp.float32)

    @pl.when(i == pl.num_programs(0) - 1)
    def _():
        pooled = sum_acc[...] / jnp.maximum(cnt_acc[...], 1.0)
        z = jnp.maximum(
            jnp.dot(pooled, wc1_ref[...], preferred_element_type=jnp.float32)
            + bc1_ref[...], 0.0)
        out_ref[...] = jnp.dot(
            z, wc2_ref[...], preferred_element_type=jnp.float32) + bc2_ref[...]


def _pool_classify(h2, bv, wc1t, bc1, wc2tp, bc2p, bn):
    n, d = h2.shape
    return pl.pallas_call(
        _pool_body,
        grid=(n // bn,),
        in_specs=[
            pl.BlockSpec((bn, d), lambda i: (i, 0)),
            pl.BlockSpec((bn, 1), lambda i: (i, 0)),
            pl.BlockSpec(wc1t.shape, lambda i: (0, 0)),
            pl.BlockSpec(bc1.shape, lambda i: (0, 0)),
            pl.BlockSpec(wc2tp.shape, lambda i: (0, 0)),
            pl.BlockSpec(bc2p.shape, lambda i: (0, 0)),
        ],
        out_specs=pl.BlockSpec((N_GRAPHS, 128), lambda i: (0, 0)),
        out_shape=jax.ShapeDtypeStruct((N_GRAPHS, 128), jnp.float32),
        scratch_shapes=[
            pltpu.VMEM((N_GRAPHS, d), jnp.float32),
            pltpu.VMEM((N_GRAPHS, d), jnp.float32),
        ],
    )(h2, bv, wc1t, bc1, wc2tp, bc2p)


def _den_pass(aux128, srcm, dstm, hpd, shared_aux):
    mesh = plsc.VectorSubcoreMesh(core_axis_name="c", subcore_axis_name="s",
                                  num_cores=NC, num_subcores=NS)

    @functools.partial(
        pl.kernel, mesh=mesh,
        compiler_params=pltpu.CompilerParams(needs_layout_passes=False),
        out_type=jax.ShapeDtypeStruct((N, NC * 128), jnp.float32),
        scratch_types=[
            pltpu.VMEM_SHARED((N, 128), jnp.float32),
            pltpu.VMEM((CK,), jnp.int32),
            pltpu.VMEM((CK,), jnp.int32),
            pltpu.VMEM((CK,), jnp.int32),
            pltpu.VMEM((CK,), jnp.int32),
            pltpu.VMEM((CK, 128), jnp.float32),
            pltpu.VMEM((CK, 128), jnp.float32),
            pltpu.VMEM((CK, 128), jnp.float32),
            pltpu.VMEM((CK, 128), jnp.float32),
            pltpu.SemaphoreType.DMA,
            pltpu.SemaphoreType.DMA,
        ],
    )
    def k(aux_h, srcm_h, dstm_h, out_h,
          acc, srcA, dstA, srcB, dstB, gsA, gdA, gsB, gdB, semA, semB):
        cidx = lax.axis_index("c")
        sidx = lax.axis_index("s")
        zero16 = jnp.zeros((16,), jnp.float32)
        zero16i = jnp.zeros((16,), jnp.int32)
        iota16 = lax.iota(jnp.int32, 16)
        nstart, ncht = _tile_rows(sidx)

        def zrow(r, _):
            for j in range(8):
                gsA[r, pl.ds(j * 16, 16)] = zero16
            return 0
        lax.fori_loop(0, FC, zrow, 0)

        def zchunk(i, _):
            r0 = pl.multiple_of(nstart + i * FC, 8)
            pltpu.sync_copy(gsA, acc.at[pl.ds(r0, FC)])
            return 0
        lax.fori_loop(0, ncht, zchunk, 0)
        plsc.subcore_barrier()

        def load(ck, src_b, dst_b, gs_b, gd_b, sem):
            e0 = pl.multiple_of(sidx * EPT + ck * CK, 8)
            pltpu.sync_copy(srcm_h.at[pl.ds(e0, CK)], src_b)
            pltpu.sync_copy(dstm_h.at[pl.ds(e0, CK)], dst_b)
            pltpu.async_copy(aux_h.at[src_b], gs_b, sem)
            pltpu.async_copy(aux_h.at[dst_b], gd_b, sem)

        def proc(src_b, dst_b, gs_b, gd_b, sem):
            pltpu.make_async_copy(aux_h.at[src_b], gs_b, sem).wait()
            pltpu.make_async_copy(aux_h.at[dst_b], gd_b, sem).wait()
            for g in range(CK // 16):
                rows = iota16 + g * 16
                for h in range(hpd):
                    if shared_aux:
                        cs, cd = zero16i, zero16i + 1
                    else:
                        cs = zero16i + (cidx * hpd + h)
                        cd = zero16i + (NC * hpd + cidx * hpd + h)
                    a = (plsc.load_gather(gs_b, [rows, cs])
                         + plsc.load_gather(gd_b, [rows, cd]))
                    a = jnp.where(a > 0, a, 0.2 * a)
                    h16 = jnp.full((16,), h, jnp.int32)
                    plsc.store_scatter(gd_b, [rows, h16], jnp.exp(a))
            pltpu.sync_copy(gd_b, acc.at[dst_b], add=True)

        load(0, srcA, dstA, gsA, gdA, semA)

        def pair(i, _):
            load(2 * i + 1, srcB, dstB, gsB, gdB, semB)
            proc(srcA, dstA, gsA, gdA, semA)
            load(2 * i + 2, srcA, dstA, gsA, gdA, semA)
            proc(srcB, dstB, gsB, gdB, semB)
            return 0
        lax.fori_loop(0, NPAIR, pair, 0)
        proc(srcA, dstA, gsA, gdA, semA)
        plsc.subcore_barrier()

        def wchunk(i, _):
            n0 = pl.multiple_of(nstart + i * FC, 8)
            c0 = pl.multiple_of(cidx * 128, 128)
            pltpu.sync_copy(acc.at[pl.ds(n0, FC)], gdA)
            pltpu.sync_copy(gdA, out_h.at[pl.ds(n0, FC), pl.ds(c0, 128)])
            return 0
        lax.fori_loop(0, ncht, wchunk, 0)

    return k(aux128, srcm, dstm)


def _edge_gat(hv, auxt3, srcm, dstm, bias, den, nseg, hpc, shared_aux):
    ndc = den.shape[0]
    mesh = plsc.VectorSubcoreMesh(core_axis_name="c", subcore_axis_name="s",
                                  num_cores=NC, num_subcores=NS)

    @functools.partial(
        pl.kernel, mesh=mesh,
        compiler_params=pltpu.CompilerParams(needs_layout_passes=False),
        out_type=jax.ShapeDtypeStruct((N, nseg * 128), jnp.float32),
        scratch_types=[
            pltpu.VMEM_SHARED((N, 128), jnp.float32),
            pltpu.VMEM((N,), jnp.float32),
            pltpu.VMEM((N,), jnp.float32),
            pltpu.VMEM((NPT,), jnp.float32),
            pltpu.VMEM((128,), jnp.float32),
            pltpu.VMEM((CK,), jnp.int32),
            pltpu.VMEM((CK,), jnp.int32),
            pltpu.VMEM((CK,), jnp.int32),
            pltpu.VMEM((CK,), jnp.float32),
            pltpu.VMEM((CK,), jnp.int32),
            pltpu.VMEM((CK,), jnp.int32),
            pltpu.VMEM((CK,), jnp.int32),
            pltpu.VMEM((CK,), jnp.float32),
            pltpu.VMEM((CK, 128), jnp.float32),
            pltpu.VMEM((CK, 128), jnp.float32),
            pltpu.VMEM((FC,), jnp.float32),
            pltpu.VMEM((FC,), jnp.float32),
            pltpu.SemaphoreType.DMA,
            pltpu.SemaphoreType.DMA,
        ],
    )
    def k(hv_h, auxt_h, srcm_h, dstm_h, bias_h, den_h, out_h,
          acc, asrc_v, adst_v, den_v, bias_v,
          srcA, dstA, idxA, wbufA, srcB, dstB, idxB, wbufB,
          gbufA, gbufB, wsbuf, invbuf, gsemA, gsemB):
        cidx = lax.axis_index("c")
        sidx = lax.axis_index("s")
        zero16 = jnp.zeros((16,), jnp.float32)
        iota16 = lax.iota(jnp.int32, 16)
        nstart, ncht = _tile_rows(sidx)

        def head_body(hd, _):
            hdg = cidx * hpc + hd
            if shared_aux:
                a_row = jnp.int32(0)
                b_row = jnp.int32(1)
                d_row = jnp.int32(0)
            else:
                a_row = hdg
                b_row = nseg + hdg
                d_row = hdg
            pltpu.sync_copy(
                den_h.at[d_row, 0,
                         pl.ds(pl.multiple_of(nstart, 8), NPT)], den_v)
            pltpu.sync_copy(auxt_h.at[a_row, 0], asrc_v)
            pltpu.sync_copy(auxt_h.at[b_row, 0], adst_v)
            pltpu.sync_copy(bias_h.at[hdg, 0], bias_v)
            bj = [bias_v[pl.ds(j * 16, 16)] for j in range(8)]

            def zrow(r, _):
                for j in range(8):
                    gbufB[r, pl.ds(j * 16, 16)] = zero16
                return 0
            lax.fori_loop(0, FC, zrow, 0)

            def zchunk(i, _):
                r0 = pl.multiple_of(nstart + i * FC, 8)
                pltpu.sync_copy(gbufB, acc.at[pl.ds(r0, FC)])
                return 0
            lax.fori_loop(0, ncht, zchunk, 0)
            plsc.subcore_barrier()

            def load(ck, src_b, dst_b, idx_b, gbuf_b, sem):
                e0 = pl.multiple_of(sidx * EPT + ck * CK, 8)
                pltpu.sync_copy(srcm_h.at[pl.ds(e0, CK)], src_b)
                pltpu.sync_copy(dstm_h.at[pl.ds(e0, CK)], dst_b)
                for g in range(CK // 16):
                    s16 = src_b[pl.ds(g * 16, 16)]
                    idx_b[pl.ds(g * 16, 16)] = s16 * nseg + hdg
                pltpu.async_copy(hv_h.at[idx_b], gbuf_b, sem)

            def proc(src_b, dst_b, idx_b, wbuf_b, gbuf_b, sem):
                for g in range(CK // 16):
                    s16 = src_b[pl.ds(g * 16, 16)]
                    d16 = dst_b[pl.ds(g * 16, 16)]
                    a = (plsc.load_gather(asrc_v, [s16])
                         + plsc.load_gather(adst_v, [d16]))
                    a = jnp.where(a > 0, a, 0.2 * a)
                    wbuf_b[pl.ds(g * 16, 16)] = jnp.exp(a)
                pltpu.make_async_copy(hv_h.at[idx_b], gbuf_b, sem).wait()

                def scale(g, _):
                    w16 = wbuf_b[pl.ds(g * 16, 16)]
                    for e in range(16):
                        w_e = w16[e]
                        row = g * 16 + e
                        for j in range(8):
                            sl = pl.ds(j * 16, 16)
                            gbuf_b[row, sl] = w_e * gbuf_b[row, sl]
                    return 0
                lax.fori_loop(0, CK // 16, scale, 0)
                pltpu.sync_copy(gbuf_b, acc.at[dst_b], add=True)

            load(0, srcA, dstA, idxA, gbufA, gsemA)

            def pair(i, _):
                load(2 * i + 1, srcB, dstB, idxB, gbufB, gsemB)
                proc(srcA, dstA, idxA, wbufA, gbufA, gsemA)
                load(2 * i + 2, srcA, dstA, idxA, gbufA, gsemA)
                proc(srcB, dstB, idxB, wbufB, gbufB, gsemB)
                return 0
            lax.fori_loop(0, NPAIR, pair, 0)
            proc(srcA, dstA, idxA, wbufA, gbufA, gsemA)
            plsc.subcore_barrier()

            def fchunk(i, _):
                n0 = pl.multiple_of(nstart + i * FC, 8)
                pltpu.sync_copy(acc.at[pl.ds(n0, FC)], gbufA)
                for g in range(FC // 16):
                    i16 = iota16 + (n0 + g * 16)
                    idxA[pl.ds(g * 16, 16)] = i16 * nseg + hdg
                cpf = pltpu.async_copy(hv_h.at[idxA], gbufB, gsemA)
                for g in range(FC // 16):
                    sl = pl.ds(g * 16, 16)
                    o16 = pl.multiple_of(n0 + g * 16, 8)
                    a = asrc_v[pl.ds(o16, 16)] + adst_v[pl.ds(o16, 16)]
                    a = jnp.where(a > 0, a, 0.2 * a)
                    ws = jnp.exp(a)
                    den16 = den_v[pl.ds(
                        pl.multiple_of(i * FC + g * 16, 8), 16)]
                    wsbuf[sl] = ws
                    invbuf[sl] = 1.0 / (den16 + ws + 1e-16)
                cpf.wait()

                def fnode(g, _):
                    ws16 = wsbuf[pl.ds(g * 16, 16)]
                    inv16 = invbuf[pl.ds(g * 16, 16)]
                    for e in range(16):
                        ws = ws16[e]
                        inv = inv16[e]
                        row = g * 16 + e
                        for j in range(8):
                            sl = pl.ds(j * 16, 16)
                            v = ((gbufA[row, sl] + ws * gbufB[row, sl]) * inv
                                 + bj[j])
                            gbufB[row, sl] = jnp.where(
                                v > 0, v, jnp.exp(v) - 1.0)
                    return 0
                lax.fori_loop(0, FC // 16, fnode, 0)
                c0 = pl.multiple_of(hdg * 128, 128)
                pltpu.sync_copy(gbufB,
                                out_h.at[pl.ds(n0, FC), pl.ds(c0, 128)])
                return 0
            lax.fori_loop(0, ncht, fchunk, 0)
            plsc.subcore_barrier()
            return 0

        lax.fori_loop(0, hpc, head_body, 0)

    return k(hv, auxt3, srcm, dstm, bias, den)


def kernel(x, W1, a_src1, a_dst1, b1, W2, a_src2, a_dst2, b2,
           Wc1, bc1, Wc2, bc2, edge_index, batch_vec):
    src = edge_index[0].astype(jnp.int32)
    dst = edge_index[1].astype(jnp.int32)

    eye1 = jnp.eye(HEADS, dtype=jnp.float32)
    bd_src1 = (a_src1[0][:, :, None] * eye1[:, None, :]).reshape(
        HEADS * HID, HEADS)
    bd_dst1 = (a_dst1[0][:, :, None] * eye1[:, None, :]).reshape(
        HEADS * HID, HEADS)
    amat1 = jnp.concatenate(
        [bd_src1, bd_dst1,
         jnp.zeros((HEADS * HID, 112), jnp.float32)], axis=1)
    amat2 = jnp.concatenate(
        [a_src2[0].T, a_dst2[0].T,
         jnp.zeros((OUT_CH, 126), jnp.float32)], axis=1)

    pad = ((0, NPAD - N), (0, 0))

    H1, AUX1 = _mm_aux(x, W1.T, amat1, bn=1000)
    denp1 = _den_pass(AUX1, src, dst, hpd=HEADS // NC, shared_aux=False)
    den1 = jnp.pad(jnp.concatenate(
        [denp1[:, 0:4], denp1[:, 128:132]], axis=1),
        pad).T.reshape(HEADS, 1, NPAD)
    auxt31 = AUX1[:, :2 * HEADS].T.reshape(2 * HEADS, 1, N)
    h1 = _edge_gat(H1.reshape(N * HEADS, HID), auxt31, src, dst,
                   b1.reshape(HEADS, 1, HID), den1,
                   nseg=HEADS, hpc=HEADS // NC, shared_aux=False)

    H2, AUX2 = _mm_aux(h1, W2.T, amat2, bn=1000)
    denp2 = _den_pass(AUX2, src, dst, hpd=1, shared_aux=True)
    den2 = jnp.pad(denp2[:, 0:1], pad).T.reshape(1, 1, NPAD)
    auxt32 = AUX2[:, :2].T.reshape(2, 1, N)
    h2 = _edge_gat(H2.reshape(N * 2, 128), auxt32, src, dst,
                   b2.reshape(2, 1, 128), den2,
                   nseg=2, hpc=1, shared_aux=True)

    bv = batch_vec.astype(jnp.int32).reshape(N, 1)
    wc2tp = jnp.concatenate(
        [Wc2.T, jnp.zeros((OUT_CH, 128 - N_CLASSES), jnp.float32)], axis=1)
    bc2p = jnp.concatenate(
        [bc2, jnp.zeros((128 - N_CLASSES,), jnp.float32)]).reshape(1, 128)
    logits_p = _pool_classify(h2, bv, Wc1.T, bc1.reshape(1, OUT_CH),
                              wc2tp, bc2p, bn=1000)
    return logits_p[:, :N_CLASSES]


# speedup vs baseline: 15.0906x; 1.0321x over previous
import functools
import jax
import jax.numpy as jnp
from jax import lax
from jax.experimental import pallas as pl
from jax.experimental.pallas import tpu as pltpu
from jax.experimental.pallas import tpu_sc as plsc

N = 10000
E = 160000
N_GRAPHS = 200
IN_CH = 256
HID = 128
HEADS = 8
OUT_CH = 256
N_CLASSES = 14

NC = 2
NS = 16
CK = 80
EPT = E // NS
NCHUNK = EPT // CK
NPAIR = (NCHUNK - 1) // 2
NPT = 640
FC = 80
NPAD = NS * NPT


def _tile_rows(sidx):
    nstart = sidx * NPT
    ncht = jnp.where(sidx < NS - 1, NPT // FC, (N - NPT * (NS - 1)) // FC)
    return nstart, ncht


def _mm_aux_body(x_ref, w_ref, a_ref, h_ref, aux_ref):
    h = jnp.dot(x_ref[...], w_ref[...], preferred_element_type=jnp.float32)
    h_ref[...] = h
    aux_ref[...] = jnp.dot(h, a_ref[...], preferred_element_type=jnp.float32)


def _mm_aux(x, wt, amat, bn):
    n, k = x.shape
    m = wt.shape[1]
    aw = amat.shape[1]
    return pl.pallas_call(
        _mm_aux_body,
        grid=(n // bn,),
        in_specs=[
            pl.BlockSpec((bn, k), lambda i: (i, 0)),
            pl.BlockSpec((k, m), lambda i: (0, 0)),
            pl.BlockSpec((m, aw), lambda i: (0, 0)),
        ],
        out_specs=[
            pl.BlockSpec((bn, m), lambda i: (i, 0)),
            pl.BlockSpec((bn, aw), lambda i: (i, 0)),
        ],
        out_shape=[
            jax.ShapeDtypeStruct((n, m), jnp.float32),
            jax.ShapeDtypeStruct((n, aw), jnp.float32),
        ],
    )(x, wt, amat)


def _pool_body(h2_ref, bv_ref, wc1_ref, bc1_ref, wc2_ref, bc2_ref,
               out_ref, sum_acc, cnt_acc):
    i = pl.program_id(0)

    @pl.when(i == 0)
    def _():
        sum_acc[...] = jnp.zeros_like(sum_acc)
        cnt_acc[...] = jnp.zeros_like(cnt_acc)

    bv = bv_ref[...]
    oh = (bv == lax.broadcasted_iota(jnp.int32, (1, N_GRAPHS), 1)
          ).astype(jnp.float32)
    h2 = h2_ref[...]
    dn = (((0,), (0,)), ((), ()))
    sum_acc[...] += lax.dot_general(oh, h2, dn,
                                    preferred_element_type=jnp.float32)
    cnt_acc[...] += lax.dot_general(oh, jnp.ones_like(h2), dn,
                                    preferred_element_type=jnp.float32)

    @pl.when(i == pl.num_programs(0) - 1)
    def _():
        pooled = sum_acc[...] / jnp.maximum(cnt_acc[...], 1.0)
        z = jnp.maximum(
            jnp.dot(pooled, wc1_ref[...], preferred_element_type=jnp.float32)
            + bc1_ref[...], 0.0)
        out_ref[...] = jnp.dot(
            z, wc2_ref[...], preferred_element_type=jnp.float32) + bc2_ref[...]


def _pool_classify(h2, bv, wc1t, bc1, wc2tp, bc2p, bn):
    n, d = h2.shape
    return pl.pallas_call(
        _pool_body,
        grid=(n // bn,),
        in_specs=[
            pl.BlockSpec((bn, d), lambda i: (i, 0)),
            pl.BlockSpec((bn, 1), lambda i: (i, 0)),
            pl.BlockSpec(wc1t.shape, lambda i: (0, 0)),
            pl.BlockSpec(bc1.shape, lambda i: (0, 0)),
            pl.BlockSpec(wc2tp.shape, lambda i: (0, 0)),
            pl.BlockSpec(bc2p.shape, lambda i: (0, 0)),
        ],
        out_specs=pl.BlockSpec((N_GRAPHS, 128), lambda i: (0, 0)),
        out_shape=jax.ShapeDtypeStruct((N_GRAPHS, 128), jnp.float32),
        scratch_shapes=[
            pltpu.VMEM((N_GRAPHS, d), jnp.float32),
            pltpu.VMEM((N_GRAPHS, d), jnp.float32),
        ],
    )(h2, bv, wc1t, bc1, wc2tp, bc2p)


def _den_pass(aux128, srcm, dstm, hpd, shared_aux):
    mesh = plsc.VectorSubcoreMesh(core_axis_name="c", subcore_axis_name="s",
                                  num_cores=NC, num_subcores=NS)

    @functools.partial(
        pl.kernel, mesh=mesh,
        compiler_params=pltpu.CompilerParams(needs_layout_passes=False),
        out_type=jax.ShapeDtypeStruct((N, NC * 128), jnp.float32),
        scratch_types=[
            pltpu.VMEM_SHARED((N, 128), jnp.float32),
            pltpu.VMEM((CK,), jnp.int32),
            pltpu.VMEM((CK,), jnp.int32),
            pltpu.VMEM((CK,), jnp.int32),
            pltpu.VMEM((CK,), jnp.int32),
            pltpu.VMEM((CK, 128), jnp.float32),
            pltpu.VMEM((CK, 128), jnp.float32),
            pltpu.VMEM((CK, 128), jnp.float32),
            pltpu.VMEM((CK, 128), jnp.float32),
            pltpu.SemaphoreType.DMA,
            pltpu.SemaphoreType.DMA,
            pltpu.SemaphoreType.DMA,
            pltpu.SemaphoreType.DMA,
        ],
    )
    def k(aux_h, srcm_h, dstm_h, out_h,
          acc, srcA, dstA, srcB, dstB, gsA, gdA, gsB, gdB,
          semA, semB, ssemA, ssemB):
        cidx = lax.axis_index("c")
        sidx = lax.axis_index("s")
        zero16 = jnp.zeros((16,), jnp.float32)
        zero16i = jnp.zeros((16,), jnp.int32)
        iota16 = lax.iota(jnp.int32, 16)
        nstart, ncht = _tile_rows(sidx)

        def zrow(r, _):
            for j in range(8):
                gsA[r, pl.ds(j * 16, 16)] = zero16
            return 0
        lax.fori_loop(0, FC, zrow, 0)

        def zchunk(i, _):
            r0 = pl.multiple_of(nstart + i * FC, 8)
            pltpu.sync_copy(gsA, acc.at[pl.ds(r0, FC)])
            return 0
        lax.fori_loop(0, ncht, zchunk, 0)
        plsc.subcore_barrier()

        def load(ck, src_b, dst_b, gs_b, gd_b, sem):
            e0 = pl.multiple_of(
                jnp.minimum(sidx * EPT + ck * CK, E - CK), 8)
            pltpu.sync_copy(srcm_h.at[pl.ds(e0, CK)], src_b)
            pltpu.sync_copy(dstm_h.at[pl.ds(e0, CK)], dst_b)
            pltpu.async_copy(aux_h.at[src_b], gs_b, sem)
            pltpu.async_copy(aux_h.at[dst_b], gd_b, sem)

        def proc(src_b, dst_b, gs_b, gd_b, sem):
            pltpu.make_async_copy(aux_h.at[src_b], gs_b, sem).wait()
            pltpu.make_async_copy(aux_h.at[dst_b], gd_b, sem).wait()
            for g in range(CK // 16):
                rows = iota16 + g * 16
                for h in range(hpd):
                    if shared_aux:
                        cs, cd = zero16i, zero16i + 1
                    else:
                        cs = zero16i + (cidx * hpd + h)
                        cd = zero16i + (NC * hpd + cidx * hpd + h)
                    a = (plsc.load_gather(gs_b, [rows, cs])
                         + plsc.load_gather(gd_b, [rows, cd]))
                    a = jnp.where(a > 0, a, 0.2 * a)
                    h16 = jnp.full((16,), h, jnp.int32)
                    plsc.store_scatter(gd_b, [rows, h16], jnp.exp(a))

        load(0, srcA, dstA, gsA, gdA, semA)
        load(1, srcB, dstB, gsB, gdB, semB)

        def pair(i, _):
            proc(srcA, dstA, gsA, gdA, semA)
            pltpu.async_copy(gdA, acc.at[dstA], ssemA, add=True)
            proc(srcB, dstB, gsB, gdB, semB)
            pltpu.async_copy(gdB, acc.at[dstB], ssemB, add=True)
            pltpu.make_async_copy(gdA, acc.at[dstA], ssemA).wait()
            load(2 * i + 2, srcA, dstA, gsA, gdA, semA)
            pltpu.make_async_copy(gdB, acc.at[dstB], ssemB).wait()
            load(2 * i + 3, srcB, dstB, gsB, gdB, semB)
            return 0
        lax.fori_loop(0, NPAIR, pair, 0)
        proc(srcA, dstA, gsA, gdA, semA)
        pltpu.sync_copy(gdA, acc.at[dstA], add=True)
        pltpu.make_async_copy(aux_h.at[srcB], gsB, semB).wait()
        pltpu.make_async_copy(aux_h.at[dstB], gdB, semB).wait()
        plsc.subcore_barrier()

        def wchunk(i, _):
            n0 = pl.multiple_of(nstart + i * FC, 8)
            c0 = pl.multiple_of(cidx * 128, 128)
            pltpu.sync_copy(acc.at[pl.ds(n0, FC)], gdA)
            pltpu.sync_copy(gdA, out_h.at[pl.ds(n0, FC), pl.ds(c0, 128)])
            return 0
        lax.fori_loop(0, ncht, wchunk, 0)

    return k(aux128, srcm, dstm)


def _edge_gat(hv, auxt3, srcm, dstm, bias, den, nseg, hpc, shared_aux):
    ndc = den.shape[0]
    mesh = plsc.VectorSubcoreMesh(core_axis_name="c", subcore_axis_name="s",
                                  num_cores=NC, num_subcores=NS)

    @functools.partial(
        pl.kernel, mesh=mesh,
        compiler_params=pltpu.CompilerParams(needs_layout_passes=False),
        out_type=jax.ShapeDtypeStruct((N, nseg * 128), jnp.float32),
        scratch_types=[
            pltpu.VMEM_SHARED((N, 128), jnp.float32),
            pltpu.VMEM((N,), jnp.float32),
            pltpu.VMEM((N,), jnp.float32),
            pltpu.VMEM((NPT,), jnp.float32),
            pltpu.VMEM((128,), jnp.float32),
            pltpu.VMEM((CK,), jnp.int32),
            pltpu.VMEM((CK,), jnp.int32),
            pltpu.VMEM((CK,), jnp.int32),
            pltpu.VMEM((CK,), jnp.float32),
            pltpu.VMEM((CK,), jnp.int32),
            pltpu.VMEM((CK,), jnp.int32),
            pltpu.VMEM((CK,), jnp.int32),
            pltpu.VMEM((CK,), jnp.float32),
            pltpu.VMEM((CK, 128), jnp.float32),
            pltpu.VMEM((CK, 128), jnp.float32),
            pltpu.VMEM((FC,), jnp.float32),
            pltpu.VMEM((FC,), jnp.float32),
            pltpu.SemaphoreType.DMA,
            pltpu.SemaphoreType.DMA,
            pltpu.SemaphoreType.DMA,
            pltpu.SemaphoreType.DMA,
        ],
    )
    def k(hv_h, auxt_h, srcm_h, dstm_h, bias_h, den_h, out_h,
          acc, asrc_v, adst_v, den_v, bias_v,
          srcA, dstA, idxA, wbufA, srcB, dstB, idxB, wbufB,
          gbufA, gbufB, wsbuf, invbuf, gsemA, gsemB, ssemA, ssemB):
        cidx = lax.axis_index("c")
        sidx = lax.axis_index("s")
        zero16 = jnp.zeros((16,), jnp.float32)
        iota16 = lax.iota(jnp.int32, 16)
        nstart, ncht = _tile_rows(sidx)

        def head_body(hd, _):
            hdg = cidx * hpc + hd
            if shared_aux:
                a_row = jnp.int32(0)
                b_row = jnp.int32(1)
                d_row = jnp.int32(0)
            else:
                a_row = hdg
                b_row = nseg + hdg
                d_row = hdg
            pltpu.sync_copy(
                den_h.at[d_row, 0,
                         pl.ds(pl.multiple_of(nstart, 8), NPT)], den_v)
            pltpu.sync_copy(auxt_h.at[a_row, 0], asrc_v)
            pltpu.sync_copy(auxt_h.at[b_row, 0], adst_v)
            pltpu.sync_copy(bias_h.at[hdg, 0], bias_v)
            bj = [bias_v[pl.ds(j * 16, 16)] for j in range(8)]

            def zrow(r, _):
                for j in range(8):
                    gbufB[r, pl.ds(j * 16, 16)] = zero16
                return 0
            lax.fori_loop(0, FC, zrow, 0)

            def zchunk(i, _):
                r0 = pl.multiple_of(nstart + i * FC, 8)
                pltpu.sync_copy(gbufB, acc.at[pl.ds(r0, FC)])
                return 0
            lax.fori_loop(0, ncht, zchunk, 0)
            plsc.subcore_barrier()

            def load(ck, src_b, dst_b, idx_b, gbuf_b, sem):
                e0 = pl.multiple_of(
                    jnp.minimum(sidx * EPT + ck * CK, E - CK), 8)
                pltpu.sync_copy(srcm_h.at[pl.ds(e0, CK)], src_b)
                pltpu.sync_copy(dstm_h.at[pl.ds(e0, CK)], dst_b)
                for g in range(CK // 16):
                    s16 = src_b[pl.ds(g * 16, 16)]
                    idx_b[pl.ds(g * 16, 16)] = s16 * nseg + hdg
                pltpu.async_copy(hv_h.at[idx_b], gbuf_b, sem)

            def proc(src_b, dst_b, idx_b, wbuf_b, gbuf_b, sem):
                for g in range(CK // 16):
                    s16 = src_b[pl.ds(g * 16, 16)]
                    d16 = dst_b[pl.ds(g * 16, 16)]
                    a = (plsc.load_gather(asrc_v, [s16])
                         + plsc.load_gather(adst_v, [d16]))
                    a = jnp.where(a > 0, a, 0.2 * a)
                    wbuf_b[pl.ds(g * 16, 16)] = jnp.exp(a)
                pltpu.make_async_copy(hv_h.at[idx_b], gbuf_b, sem).wait()

                def scale(g, _):
                    w16 = wbuf_b[pl.ds(g * 16, 16)]
                    for e in range(16):
                        w_e = w16[e]
                        row = g * 16 + e
                        for j in range(8):
                            sl = pl.ds(j * 16, 16)
                            gbuf_b[row, sl] = w_e * gbuf_b[row, sl]
                    return 0
                lax.fori_loop(0, CK // 16, scale, 0)

            load(0, srcA, dstA, idxA, gbufA, gsemA)
            load(1, srcB, dstB, idxB, gbufB, gsemB)

            def pair(i, _):
                proc(srcA, dstA, idxA, wbufA, gbufA, gsemA)
                pltpu.async_copy(gbufA, acc.at[dstA], ssemA, add=True)
                proc(srcB, dstB, idxB, wbufB, gbufB, gsemB)
                pltpu.async_copy(gbufB, acc.at[dstB], ssemB, add=True)
                pltpu.make_async_copy(gbufA, acc.at[dstA], ssemA).wait()
                load(2 * i + 2, srcA, dstA, idxA, gbufA, gsemA)
                pltpu.make_async_copy(gbufB, acc.at[dstB], ssemB).wait()
                load(2 * i + 3, srcB, dstB, idxB, gbufB, gsemB)
                return 0
            lax.fori_loop(0, NPAIR, pair, 0)
            proc(srcA, dstA, idxA, wbufA, gbufA, gsemA)
            pltpu.sync_copy(gbufA, acc.at[dstA], add=True)
            pltpu.make_async_copy(hv_h.at[idxB], gbufB, gsemB).wait()
            plsc.subcore_barrier()

            def fchunk(i, _):
                n0 = pl.multiple_of(nstart + i * FC, 8)
                pltpu.sync_copy(acc.at[pl.ds(n0, FC)], gbufA)
                for g in range(FC // 16):
                    i16 = iota16 + (n0 + g * 16)
                    idxA[pl.ds(g * 16, 16)] = i16 * nseg + hdg
                cpf = pltpu.async_copy(hv_h.at[idxA], gbufB, gsemA)
                for g in range(FC // 16):
                    sl = pl.ds(g * 16, 16)
                    o16 = pl.multiple_of(n0 + g * 16, 8)
                    a = asrc_v[pl.ds(o16, 16)] + adst_v[pl.ds(o16, 16)]
                    a = jnp.where(a > 0, a, 0.2 * a)
                    ws = jnp.exp(a)
                    den16 = den_v[pl.ds(
                        pl.multiple_of(i * FC + g * 16, 8), 16)]
                    wsbuf[sl] = ws
                    invbuf[sl] = 1.0 / (den16 + ws + 1e-16)
                cpf.wait()

                def fnode(g, _):
                    ws16 = wsbuf[pl.ds(g * 16, 16)]
                    inv16 = invbuf[pl.ds(g * 16, 16)]
                    for e in range(16):
                        ws = ws16[e]
                        inv = inv16[e]
                        row = g * 16 + e
                        for j in range(8):
                            sl = pl.ds(j * 16, 16)
                            v = ((gbufA[row, sl] + ws * gbufB[row, sl]) * inv
                                 + bj[j])
                            gbufB[row, sl] = jnp.where(
                                v > 0, v, jnp.exp(v) - 1.0)
                    return 0
                lax.fori_loop(0, FC // 16, fnode, 0)
                c0 = pl.multiple_of(hdg * 128, 128)
                pltpu.sync_copy(gbufB,
                                out_h.at[pl.ds(n0, FC), pl.ds(c0, 128)])
                return 0
            lax.fori_loop(0, ncht, fchunk, 0)
            plsc.subcore_barrier()
            return 0

        lax.fori_loop(0, hpc, head_body, 0)

    return k(hv, auxt3, srcm, dstm, bias, den)


def kernel(x, W1, a_src1, a_dst1, b1, W2, a_src2, a_dst2, b2,
           Wc1, bc1, Wc2, bc2, edge_index, batch_vec):
    src = edge_index[0].astype(jnp.int32)
    dst = edge_index[1].astype(jnp.int32)

    eye1 = jnp.eye(HEADS, dtype=jnp.float32)
    bd_src1 = (a_src1[0][:, :, None] * eye1[:, None, :]).reshape(
        HEADS * HID, HEADS)
    bd_dst1 = (a_dst1[0][:, :, None] * eye1[:, None, :]).reshape(
        HEADS * HID, HEADS)
    amat1 = jnp.concatenate(
        [bd_src1, bd_dst1,
         jnp.zeros((HEADS * HID, 112), jnp.float32)], axis=1)
    amat2 = jnp.concatenate(
        [a_src2[0].T, a_dst2[0].T,
         jnp.zeros((OUT_CH, 126), jnp.float32)], axis=1)

    pad = ((0, NPAD - N), (0, 0))

    H1, AUX1 = _mm_aux(x, W1.T, amat1, bn=1000)
    denp1 = _den_pass(AUX1, src, dst, hpd=HEADS // NC, shared_aux=False)
    den1 = jnp.pad(jnp.concatenate(
        [denp1[:, 0:4], denp1[:, 128:132]], axis=1),
        pad).T.reshape(HEADS, 1, NPAD)
    auxt31 = AUX1[:, :2 * HEADS].T.reshape(2 * HEADS, 1, N)
    h1 = _edge_gat(H1.reshape(N * HEADS, HID), auxt31, src, dst,
                   b1.reshape(HEADS, 1, HID), den1,
                   nseg=HEADS, hpc=HEADS // NC, shared_aux=False)

    H2, AUX2 = _mm_aux(h1, W2.T, amat2, bn=1000)
    denp2 = _den_pass(AUX2, src, dst, hpd=1, shared_aux=True)
    den2 = jnp.pad(denp2[:, 0:1], pad).T.reshape(1, 1, NPAD)
    auxt32 = AUX2[:, :2].T.reshape(2, 1, N)
    h2 = _edge_gat(H2.reshape(N * 2, 128), auxt32, src, dst,
                   b2.reshape(2, 1, 128), den2,
                   nseg=2, hpc=1, shared_aux=True)

    bv = batch_vec.astype(jnp.int32).reshape(N, 1)
    wc2tp = jnp.concatenate(
        [Wc2.T, jnp.zeros((OUT_CH, 128 - N_CLASSES), jnp.float32)], axis=1)
    bc2p = jnp.concatenate(
        [bc2, jnp.zeros((128 - N_CLASSES,), jnp.float32)]).reshape(1, 128)
    logits_p = _pool_classify(h2, bv, Wc1.T, bc1.reshape(1, OUT_CH),
                              wc2tp, bc2p, bn=1000)
    return logits_p[:, :N_CLASSES]


# speedup vs baseline: 19.4572x; 1.2894x over previous
import functools
import jax
import jax.numpy as jnp
from jax import lax
from jax.experimental import pallas as pl
from jax.experimental.pallas import tpu as pltpu
from jax.experimental.pallas import tpu_sc as plsc

N = 10000
E = 160000
N_GRAPHS = 200
IN_CH = 256
HID = 128
HEADS = 8
OUT_CH = 256
N_CLASSES = 14

NC = 2
NS = 16
CK = 80
EPT = E // NS
NCHUNK = EPT // CK
NPAIR = (NCHUNK - 1) // 2
NPT = 640
FC = 80
NPAD = NS * NPT


def _tile_rows(sidx):
    nstart = sidx * NPT
    ncht = jnp.where(sidx < NS - 1, NPT // FC, (N - NPT * (NS - 1)) // FC)
    return nstart, ncht


def _mm_aux_body(x_ref, w_ref, a_ref, h_ref, aux_ref):
    h = jnp.dot(x_ref[...], w_ref[...], preferred_element_type=jnp.float32)
    h_ref[...] = h
    aux_ref[...] = jnp.dot(h, a_ref[...], preferred_element_type=jnp.float32)


def _mm_aux(x, wt, amat, bn):
    n, k = x.shape
    m = wt.shape[1]
    aw = amat.shape[1]
    return pl.pallas_call(
        _mm_aux_body,
        grid=(n // bn,),
        in_specs=[
            pl.BlockSpec((bn, k), lambda i: (i, 0)),
            pl.BlockSpec((k, m), lambda i: (0, 0)),
            pl.BlockSpec((m, aw), lambda i: (0, 0)),
        ],
        out_specs=[
            pl.BlockSpec((bn, m), lambda i: (i, 0)),
            pl.BlockSpec((bn, aw), lambda i: (i, 0)),
        ],
        out_shape=[
            jax.ShapeDtypeStruct((n, m), jnp.float32),
            jax.ShapeDtypeStruct((n, aw), jnp.float32),
        ],
    )(x, wt, amat)


def _pool_body(h2_ref, bv_ref, wc1_ref, bc1_ref, wc2_ref, bc2_ref,
               out_ref, sum_acc, cnt_acc):
    i = pl.program_id(0)

    @pl.when(i == 0)
    def _():
        sum_acc[...] = jnp.zeros_like(sum_acc)
        cnt_acc[...] = jnp.zeros_like(cnt_acc)

    bv = bv_ref[...]
    oh = (bv == lax.broadcasted_iota(jnp.int32, (1, N_GRAPHS), 1)
          ).astype(jnp.float32)
    h2 = h2_ref[...]
    dn = (((0,), (0,)), ((), ()))
    sum_acc[...] += lax.dot_general(oh, h2, dn,
                                    preferred_element_type=jnp.float32)
    cnt_acc[...] += lax.dot_general(oh, jnp.ones_like(h2), dn,
                                    preferred_element_type=jnp.float32)

    @pl.when(i == pl.num_programs(0) - 1)
    def _():
        pooled = sum_acc[...] / jnp.maximum(cnt_acc[...], 1.0)
        z = jnp.maximum(
            jnp.dot(pooled, wc1_ref[...], preferred_element_type=jnp.float32)
            + bc1_ref[...], 0.0)
        out_ref[...] = jnp.dot(
            z, wc2_ref[...], preferred_element_type=jnp.float32) + bc2_ref[...]


def _pool_classify(h2, bv, wc1t, bc1, wc2tp, bc2p, bn):
    n, d = h2.shape
    return pl.pallas_call(
        _pool_body,
        grid=(n // bn,),
        in_specs=[
            pl.BlockSpec((bn, d), lambda i: (i, 0)),
            pl.BlockSpec((bn, 1), lambda i: (i, 0)),
            pl.BlockSpec(wc1t.shape, lambda i: (0, 0)),
            pl.BlockSpec(bc1.shape, lambda i: (0, 0)),
            pl.BlockSpec(wc2tp.shape, lambda i: (0, 0)),
            pl.BlockSpec(bc2p.shape, lambda i: (0, 0)),
        ],
        out_specs=pl.BlockSpec((N_GRAPHS, 128), lambda i: (0, 0)),
        out_shape=jax.ShapeDtypeStruct((N_GRAPHS, 128), jnp.float32),
        scratch_shapes=[
            pltpu.VMEM((N_GRAPHS, d), jnp.float32),
            pltpu.VMEM((N_GRAPHS, d), jnp.float32),
        ],
    )(h2, bv, wc1t, bc1, wc2tp, bc2p)


def _den_pass(aux128, sd, hpd, shared_aux):
    mesh = plsc.VectorSubcoreMesh(core_axis_name="c", subcore_axis_name="s",
                                  num_cores=NC, num_subcores=NS)

    @functools.partial(
        pl.kernel, mesh=mesh,
        compiler_params=pltpu.CompilerParams(needs_layout_passes=False),
        out_type=jax.ShapeDtypeStruct((N, NC * 128), jnp.float32),
        scratch_types=[
            pltpu.VMEM_SHARED((N, 128), jnp.float32),
            pltpu.VMEM((2 * CK,), jnp.int32),
            pltpu.VMEM((CK,), jnp.int32),
            pltpu.VMEM((2 * CK,), jnp.int32),
            pltpu.VMEM((CK,), jnp.int32),
            pltpu.VMEM((CK, 128), jnp.float32),
            pltpu.VMEM((CK, 128), jnp.float32),
            pltpu.VMEM((CK, 128), jnp.float32),
            pltpu.VMEM((CK, 128), jnp.float32),
            pltpu.SemaphoreType.DMA,
            pltpu.SemaphoreType.DMA,
            pltpu.SemaphoreType.DMA,
            pltpu.SemaphoreType.DMA,
            pltpu.SemaphoreType.DMA,
            pltpu.SemaphoreType.DMA,
        ],
    )
    def k(aux_h, sd_h, out_h,
          acc, sdA, dstA, sdB, dstB, gsA, gdA, gsB, gdB,
          semA, semB, ssemA, ssemB, sdsemA, sdsemB):
        cidx = lax.axis_index("c")
        sidx = lax.axis_index("s")
        zero16 = jnp.zeros((16,), jnp.float32)
        zero16i = jnp.zeros((16,), jnp.int32)
        iota16 = lax.iota(jnp.int32, 16)
        nstart, ncht = _tile_rows(sidx)

        def zrow(r, _):
            for j in range(8):
                gsA[r, pl.ds(j * 16, 16)] = zero16
            return 0
        lax.fori_loop(0, FC, zrow, 0)

        def zchunk(i, _):
            r0 = pl.multiple_of(nstart + i * FC, 8)
            pltpu.sync_copy(gsA, acc.at[pl.ds(r0, FC)])
            return 0
        lax.fori_loop(0, ncht, zchunk, 0)
        plsc.subcore_barrier()

        def fetch(ck, sd_b, sdsem):
            g0 = jnp.minimum(sidx * NCHUNK + ck, NS * NCHUNK - 1)
            off = pl.multiple_of(g0 * (2 * CK), 8)
            pltpu.async_copy(sd_h.at[pl.ds(off, 2 * CK)], sd_b, sdsem)

        def gstart(sd_b, dst_b, gs_b, gd_b, sdsem, gsem):
            pltpu.make_async_copy(
                sd_h.at[pl.ds(0, 2 * CK)], sd_b, sdsem).wait()
            for g in range(CK // 16):
                dst_b[pl.ds(g * 16, 16)] = sd_b[pl.ds(CK + g * 16, 16)]
            pltpu.async_copy(aux_h.at[sd_b.at[pl.ds(0, CK)]], gs_b, gsem)
            pltpu.async_copy(aux_h.at[dst_b], gd_b, gsem)

        def proc(nxt, sd_b, dst_b, gs_b, gd_b, sdsem, gsem):
            pltpu.make_async_copy(aux_h.at[dst_b], gs_b, gsem).wait()
            pltpu.make_async_copy(aux_h.at[dst_b], gd_b, gsem).wait()
            for g in range(CK // 16):
                rows = iota16 + g * 16
                for h in range(hpd):
                    if shared_aux:
                        cs, cd = zero16i, zero16i + 1
                    else:
                        cs = zero16i + (cidx * hpd + h)
                        cd = zero16i + (NC * hpd + cidx * hpd + h)
                    a = (plsc.load_gather(gs_b, [rows, cs])
                         + plsc.load_gather(gd_b, [rows, cd]))
                    a = jnp.where(a > 0, a, 0.2 * a)
                    h16 = jnp.full((16,), h, jnp.int32)
                    plsc.store_scatter(gd_b, [rows, h16], jnp.exp(a))
            fetch(nxt, sd_b, sdsem)

        fetch(0, sdA, sdsemA)
        fetch(1, sdB, sdsemB)
        gstart(sdA, dstA, gsA, gdA, sdsemA, semA)
        gstart(sdB, dstB, gsB, gdB, sdsemB, semB)

        def pair(i, _):
            proc(2 * i + 2, sdA, dstA, gsA, gdA, sdsemA, semA)
            pltpu.async_copy(gdA, acc.at[dstA], ssemA, add=True)
            proc(2 * i + 3, sdB, dstB, gsB, gdB, sdsemB, semB)
            pltpu.async_copy(gdB, acc.at[dstB], ssemB, add=True)
            pltpu.make_async_copy(gdA, acc.at[dstA], ssemA).wait()
            gstart(sdA, dstA, gsA, gdA, sdsemA, semA)
            pltpu.make_async_copy(gdB, acc.at[dstB], ssemB).wait()
            gstart(sdB, dstB, gsB, gdB, sdsemB, semB)
            return 0
        lax.fori_loop(0, NPAIR, pair, 0)
        proc(NCHUNK, sdA, dstA, gsA, gdA, sdsemA, semA)
        pltpu.sync_copy(gdA, acc.at[dstA], add=True)
        pltpu.make_async_copy(aux_h.at[dstB], gsB, semB).wait()
        pltpu.make_async_copy(aux_h.at[dstB], gdB, semB).wait()
        pltpu.make_async_copy(sd_h.at[pl.ds(0, 2 * CK)], sdA, sdsemA).wait()
        plsc.subcore_barrier()

        def wchunk(i, _):
            n0 = pl.multiple_of(nstart + i * FC, 8)
            c0 = pl.multiple_of(cidx * 128, 128)
            pltpu.sync_copy(acc.at[pl.ds(n0, FC)], gdA)
            pltpu.sync_copy(gdA, out_h.at[pl.ds(n0, FC), pl.ds(c0, 128)])
            return 0
        lax.fori_loop(0, ncht, wchunk, 0)

    return k(aux128, sd)


def _edge_gat(hv, auxt3, sd, bias, den, nseg, hpc, shared_aux):
    ndc = den.shape[0]
    mesh = plsc.VectorSubcoreMesh(core_axis_name="c", subcore_axis_name="s",
                                  num_cores=NC, num_subcores=NS)

    @functools.partial(
        pl.kernel, mesh=mesh,
        compiler_params=pltpu.CompilerParams(needs_layout_passes=False),
        out_type=jax.ShapeDtypeStruct((N, nseg * 128), jnp.float32),
        scratch_types=[
            pltpu.VMEM_SHARED((N, 128), jnp.float32),
            pltpu.VMEM((N,), jnp.float32),
            pltpu.VMEM((N,), jnp.float32),
            pltpu.VMEM((NPT,), jnp.float32),
            pltpu.VMEM((128,), jnp.float32),
            pltpu.VMEM((2 * CK,), jnp.int32),
            pltpu.VMEM((CK,), jnp.int32),
            pltpu.VMEM((CK,), jnp.int32),
            pltpu.VMEM((CK,), jnp.float32),
            pltpu.VMEM((2 * CK,), jnp.int32),
            pltpu.VMEM((CK,), jnp.int32),
            pltpu.VMEM((CK,), jnp.int32),
            pltpu.VMEM((CK,), jnp.float32),
            pltpu.VMEM((CK, 128), jnp.float32),
            pltpu.VMEM((CK, 128), jnp.float32),
            pltpu.VMEM((FC,), jnp.float32),
            pltpu.VMEM((FC,), jnp.float32),
            pltpu.SemaphoreType.DMA,
            pltpu.SemaphoreType.DMA,
            pltpu.SemaphoreType.DMA,
            pltpu.SemaphoreType.DMA,
            pltpu.SemaphoreType.DMA,
            pltpu.SemaphoreType.DMA,
        ],
    )
    def k(hv_h, auxt_h, sd_h, bias_h, den_h, out_h,
          acc, asrc_v, adst_v, den_v, bias_v,
          sdA, dstA, idxA, wbufA, sdB, dstB, idxB, wbufB,
          gbufA, gbufB, wsbuf, invbuf,
          gsemA, gsemB, ssemA, ssemB, sdsemA, sdsemB):
        cidx = lax.axis_index("c")
        sidx = lax.axis_index("s")
        zero16 = jnp.zeros((16,), jnp.float32)
        iota16 = lax.iota(jnp.int32, 16)
        nstart, ncht = _tile_rows(sidx)

        def head_body(hd, _):
            hdg = cidx * hpc + hd
            if shared_aux:
                a_row = jnp.int32(0)
                b_row = jnp.int32(1)
                d_row = jnp.int32(0)
            else:
                a_row = hdg
                b_row = nseg + hdg
                d_row = hdg
            pltpu.sync_copy(
                den_h.at[d_row, 0,
                         pl.ds(pl.multiple_of(nstart, 8), NPT)], den_v)
            pltpu.sync_copy(auxt_h.at[a_row, 0], asrc_v)
            pltpu.sync_copy(auxt_h.at[b_row, 0], adst_v)
            pltpu.sync_copy(bias_h.at[hdg, 0], bias_v)
            bj = [bias_v[pl.ds(j * 16, 16)] for j in range(8)]

            def zrow(r, _):
                for j in range(8):
                    gbufB[r, pl.ds(j * 16, 16)] = zero16
                return 0
            lax.fori_loop(0, FC, zrow, 0)

            def zchunk(i, _):
                r0 = pl.multiple_of(nstart + i * FC, 8)
                pltpu.sync_copy(gbufB, acc.at[pl.ds(r0, FC)])
                return 0
            lax.fori_loop(0, ncht, zchunk, 0)
            plsc.subcore_barrier()

            def fetch(ck, sd_b, sdsem):
                g0 = jnp.minimum(sidx * NCHUNK + ck, NS * NCHUNK - 1)
                off = pl.multiple_of(g0 * (2 * CK), 8)
                pltpu.async_copy(sd_h.at[pl.ds(off, 2 * CK)], sd_b, sdsem)

            def gstart(sd_b, dst_b, idx_b, gbuf_b, sdsem, gsem):
                pltpu.make_async_copy(
                    sd_h.at[pl.ds(0, 2 * CK)], sd_b, sdsem).wait()
                for g in range(CK // 16):
                    s16 = sd_b[pl.ds(g * 16, 16)]
                    idx_b[pl.ds(g * 16, 16)] = s16 * nseg + hdg
                    dst_b[pl.ds(g * 16, 16)] = sd_b[pl.ds(CK + g * 16, 16)]
                pltpu.async_copy(hv_h.at[idx_b], gbuf_b, gsem)

            def proc(nxt, sd_b, dst_b, idx_b, wbuf_b, gbuf_b, sdsem, sem):
                for g in range(CK // 16):
                    s16 = sd_b[pl.ds(g * 16, 16)]
                    d16 = dst_b[pl.ds(g * 16, 16)]
                    a = (plsc.load_gather(asrc_v, [s16])
                         + plsc.load_gather(adst_v, [d16]))
                    a = jnp.where(a > 0, a, 0.2 * a)
                    wbuf_b[pl.ds(g * 16, 16)] = jnp.exp(a)
                fetch(nxt, sd_b, sdsem)
                pltpu.make_async_copy(hv_h.at[idx_b], gbuf_b, sem).wait()

                def scale(g, _):
                    w16 = wbuf_b[pl.ds(g * 16, 16)]
                    for e in range(16):
                        w_e = w16[e]
                        row = g * 16 + e
                        for j in range(8):
                            sl = pl.ds(j * 16, 16)
                            gbuf_b[row, sl] = w_e * gbuf_b[row, sl]
                    return 0
                lax.fori_loop(0, CK // 16, scale, 0)

            fetch(0, sdA, sdsemA)
            fetch(1, sdB, sdsemB)
            gstart(sdA, dstA, idxA, gbufA, sdsemA, gsemA)
            gstart(sdB, dstB, idxB, gbufB, sdsemB, gsemB)

            def pair(i, _):
                proc(2 * i + 2, sdA, dstA, idxA, wbufA, gbufA, sdsemA, gsemA)
                pltpu.async_copy(gbufA, acc.at[dstA], ssemA, add=True)
                proc(2 * i + 3, sdB, dstB, idxB, wbufB, gbufB, sdsemB, gsemB)
                pltpu.async_copy(gbufB, acc.at[dstB], ssemB, add=True)
                pltpu.make_async_copy(gbufA, acc.at[dstA], ssemA).wait()
                gstart(sdA, dstA, idxA, gbufA, sdsemA, gsemA)
                pltpu.make_async_copy(gbufB, acc.at[dstB], ssemB).wait()
                gstart(sdB, dstB, idxB, gbufB, sdsemB, gsemB)
                return 0
            lax.fori_loop(0, NPAIR, pair, 0)
            proc(NCHUNK, sdA, dstA, idxA, wbufA, gbufA, sdsemA, gsemA)
            pltpu.sync_copy(gbufA, acc.at[dstA], add=True)
            pltpu.make_async_copy(hv_h.at[idxB], gbufB, gsemB).wait()
            pltpu.make_async_copy(
                sd_h.at[pl.ds(0, 2 * CK)], sdA, sdsemA).wait()
            plsc.subcore_barrier()

            def fchunk(i, _):
                n0 = pl.multiple_of(nstart + i * FC, 8)
                pltpu.sync_copy(acc.at[pl.ds(n0, FC)], gbufA)
                for g in range(FC // 16):
                    i16 = iota16 + (n0 + g * 16)
                    idxA[pl.ds(g * 16, 16)] = i16 * nseg + hdg
                cpf = pltpu.async_copy(hv_h.at[idxA], gbufB, gsemA)
                for g in range(FC // 16):
                    sl = pl.ds(g * 16, 16)
                    o16 = pl.multiple_of(n0 + g * 16, 8)
                    a = asrc_v[pl.ds(o16, 16)] + adst_v[pl.ds(o16, 16)]
                    a = jnp.where(a > 0, a, 0.2 * a)
                    ws = jnp.exp(a)
                    den16 = den_v[pl.ds(
                        pl.multiple_of(i * FC + g * 16, 8), 16)]
                    wsbuf[sl] = ws
                    invbuf[sl] = 1.0 / (den16 + ws + 1e-16)
                cpf.wait()

                def fnode(g, _):
                    ws16 = wsbuf[pl.ds(g * 16, 16)]
                    inv16 = invbuf[pl.ds(g * 16, 16)]
                    for e in range(16):
                        ws = ws16[e]
                        inv = inv16[e]
                        row = g * 16 + e
                        for j in range(8):
                            sl = pl.ds(j * 16, 16)
                            v = ((gbufA[row, sl] + ws * gbufB[row, sl]) * inv
                                 + bj[j])
                            gbufB[row, sl] = jnp.where(
                                v > 0, v, jnp.exp(v) - 1.0)
                    return 0
                lax.fori_loop(0, FC // 16, fnode, 0)
                c0 = pl.multiple_of(hdg * 128, 128)
                pltpu.sync_copy(gbufB,
                                out_h.at[pl.ds(n0, FC), pl.ds(c0, 128)])
                return 0
            lax.fori_loop(0, ncht, fchunk, 0)
            plsc.subcore_barrier()
            return 0

        lax.fori_loop(0, hpc, head_body, 0)

    return k(hv, auxt3, sd, bias, den)


def kernel(x, W1, a_src1, a_dst1, b1, W2, a_src2, a_dst2, b2,
           Wc1, bc1, Wc2, bc2, edge_index, batch_vec):
    src = edge_index[0].astype(jnp.int32)
    dst = edge_index[1].astype(jnp.int32)
    sd = jnp.concatenate(
        [src.reshape(E // CK, CK), dst.reshape(E // CK, CK)],
        axis=1).reshape(-1)

    eye1 = jnp.eye(HEADS, dtype=jnp.float32)
    bd_src1 = (a_src1[0][:, :, None] * eye1[:, None, :]).reshape(
        HEADS * HID, HEADS)
    bd_dst1 = (a_dst1[0][:, :, None] * eye1[:, None, :]).reshape(
        HEADS * HID, HEADS)
    amat1 = jnp.concatenate(
        [bd_src1, bd_dst1,
         jnp.zeros((HEADS * HID, 112), jnp.float32)], axis=1)
    amat2 = jnp.concatenate(
        [a_src2[0].T, a_dst2[0].T,
         jnp.zeros((OUT_CH, 126), jnp.float32)], axis=1)

    pad = ((0, NPAD - N), (0, 0))

    H1, AUX1 = _mm_aux(x, W1.T, amat1, bn=1000)
    denp1 = _den_pass(AUX1, sd, hpd=HEADS // NC, shared_aux=False)
    den1 = jnp.pad(jnp.concatenate(
        [denp1[:, 0:4], denp1[:, 128:132]], axis=1),
        pad).T.reshape(HEADS, 1, NPAD)
    auxt31 = AUX1[:, :2 * HEADS].T.reshape(2 * HEADS, 1, N)
    h1 = _edge_gat(H1.reshape(N * HEADS, HID), auxt31, sd,
                   b1.reshape(HEADS, 1, HID), den1,
                   nseg=HEADS, hpc=HEADS // NC, shared_aux=False)

    H2, AUX2 = _mm_aux(h1, W2.T, amat2, bn=1000)
    denp2 = _den_pass(AUX2, sd, hpd=1, shared_aux=True)
    den2 = jnp.pad(denp2[:, 0:1], pad).T.reshape(1, 1, NPAD)
    auxt32 = AUX2[:, :2].T.reshape(2, 1, N)
    h2 = _edge_gat(H2.reshape(N * 2, 128), auxt32, sd,
                   b2.reshape(2, 1, 128), den2,
                   nseg=2, hpc=1, shared_aux=True)

    bv = batch_vec.astype(jnp.int32).reshape(N, 1)
    wc2tp = jnp.concatenate(
        [Wc2.T, jnp.zeros((OUT_CH, 128 - N_CLASSES), jnp.float32)], axis=1)
    bc2p = jnp.concatenate(
        [bc2, jnp.zeros((128 - N_CLASSES,), jnp.float32)]).reshape(1, 128)
    logits_p = _pool_classify(h2, bv, Wc1.T, bc1.reshape(1, OUT_CH),
                              wc2tp, bc2p, bn=1000)
    return logits_p[:, :N_CLASSES]


# speedup vs baseline: 20.7281x; 1.0653x over previous
import functools
import jax
import jax.numpy as jnp
from jax import lax
from jax.experimental import pallas as pl
from jax.experimental.pallas import tpu as pltpu
from jax.experimental.pallas import tpu_sc as plsc

N = 10000
E = 160000
N_GRAPHS = 200
IN_CH = 256
HID = 128
HEADS = 8
OUT_CH = 256
N_CLASSES = 14

NC = 2
NS = 16
CK = 80
EPT = E // NS
NCHUNK = EPT // CK
NPAIR = (NCHUNK - 1) // 2
NPT = 640
FC = 80
NPAD = NS * NPT


def _tile_rows(sidx):
    nstart = sidx * NPT
    ncht = jnp.where(sidx < NS - 1, NPT // FC, (N - NPT * (NS - 1)) // FC)
    return nstart, ncht


def _mm_aux_body(x_ref, w_ref, a_ref, h_ref, aux_ref):
    h = jnp.dot(x_ref[...], w_ref[...], preferred_element_type=jnp.float32)
    h_ref[...] = h
    aux_ref[...] = jnp.dot(h, a_ref[...], preferred_element_type=jnp.float32)


def _mm_aux(x, wt, amat, bn):
    n, k = x.shape
    m = wt.shape[1]
    aw = amat.shape[1]
    return pl.pallas_call(
        _mm_aux_body,
        grid=(n // bn,),
        in_specs=[
            pl.BlockSpec((bn, k), lambda i: (i, 0)),
            pl.BlockSpec((k, m), lambda i: (0, 0)),
            pl.BlockSpec((m, aw), lambda i: (0, 0)),
        ],
        out_specs=[
            pl.BlockSpec((bn, m), lambda i: (i, 0)),
            pl.BlockSpec((bn, aw), lambda i: (i, 0)),
        ],
        out_shape=[
            jax.ShapeDtypeStruct((n, m), jnp.float32),
            jax.ShapeDtypeStruct((n, aw), jnp.float32),
        ],
    )(x, wt, amat)


def _pool_body(h2_ref, bv_ref, wc1_ref, bc1_ref, wc2_ref, bc2_ref,
               out_ref, sum_acc, cnt_acc):
    i = pl.program_id(0)

    @pl.when(i == 0)
    def _():
        sum_acc[...] = jnp.zeros_like(sum_acc)
        cnt_acc[...] = jnp.zeros_like(cnt_acc)

    bv = bv_ref[...]
    oh = (bv == lax.broadcasted_iota(jnp.int32, (1, N_GRAPHS), 1)
          ).astype(jnp.float32)
    h2 = h2_ref[...]
    dn = (((0,), (0,)), ((), ()))
    sum_acc[...] += lax.dot_general(oh, h2, dn,
                                    preferred_element_type=jnp.float32)
    cnt_acc[...] += lax.dot_general(oh, jnp.ones_like(h2), dn,
                                    preferred_element_type=jnp.float32)

    @pl.when(i == pl.num_programs(0) - 1)
    def _():
        pooled = sum_acc[...] / jnp.maximum(cnt_acc[...], 1.0)
        z = jnp.maximum(
            jnp.dot(pooled, wc1_ref[...], preferred_element_type=jnp.float32)
            + bc1_ref[...], 0.0)
        out_ref[...] = jnp.dot(
            z, wc2_ref[...], preferred_element_type=jnp.float32) + bc2_ref[...]


def _pool_classify(h2, bv, wc1t, bc1, wc2tp, bc2p, bn):
    n, d = h2.shape
    return pl.pallas_call(
        _pool_body,
        grid=(n // bn,),
        in_specs=[
            pl.BlockSpec((bn, d), lambda i: (i, 0)),
            pl.BlockSpec((bn, 1), lambda i: (i, 0)),
            pl.BlockSpec(wc1t.shape, lambda i: (0, 0)),
            pl.BlockSpec(bc1.shape, lambda i: (0, 0)),
            pl.BlockSpec(wc2tp.shape, lambda i: (0, 0)),
            pl.BlockSpec(bc2p.shape, lambda i: (0, 0)),
        ],
        out_specs=pl.BlockSpec((N_GRAPHS, 128), lambda i: (0, 0)),
        out_shape=jax.ShapeDtypeStruct((N_GRAPHS, 128), jnp.float32),
        scratch_shapes=[
            pltpu.VMEM((N_GRAPHS, d), jnp.float32),
            pltpu.VMEM((N_GRAPHS, d), jnp.float32),
        ],
    )(h2, bv, wc1t, bc1, wc2tp, bc2p)


def _den_pass(aux128, sd, hpd, shared_aux):
    mesh = plsc.VectorSubcoreMesh(core_axis_name="c", subcore_axis_name="s",
                                  num_cores=NC, num_subcores=NS)

    @functools.partial(
        pl.kernel, mesh=mesh,
        compiler_params=pltpu.CompilerParams(needs_layout_passes=False),
        out_type=jax.ShapeDtypeStruct((NC, NS, 1, hpd * N), jnp.float32),
        scratch_types=[
            pltpu.VMEM((hpd * N,), jnp.float32),
            pltpu.VMEM((2 * CK,), jnp.int32),
            pltpu.VMEM((CK,), jnp.int32),
            pltpu.VMEM((2 * CK,), jnp.int32),
            pltpu.VMEM((CK,), jnp.int32),
            pltpu.VMEM((CK, 128), jnp.float32),
            pltpu.VMEM((CK, 128), jnp.float32),
            pltpu.VMEM((CK, 128), jnp.float32),
            pltpu.VMEM((CK, 128), jnp.float32),
            pltpu.SemaphoreType.DMA,
            pltpu.SemaphoreType.DMA,
            pltpu.SemaphoreType.DMA,
            pltpu.SemaphoreType.DMA,
        ],
    )
    def k(aux_h, sd_h, out_h,
          den_l, sdA, dstA, sdB, dstB, gsA, gdA, gsB, gdB,
          semA, semB, sdsemA, sdsemB):
        cidx = lax.axis_index("c")
        sidx = lax.axis_index("s")
        zero16 = jnp.zeros((16,), jnp.float32)
        zero16i = jnp.zeros((16,), jnp.int32)
        iota16 = lax.iota(jnp.int32, 16)

        def zloop(r, _):
            den_l[pl.ds(r * 16, 16)] = zero16
            return 0
        lax.fori_loop(0, hpd * N // 16, zloop, 0)

        def fetch(ck, sd_b, sdsem):
            g0 = jnp.minimum(sidx * NCHUNK + ck, NS * NCHUNK - 1)
            off = pl.multiple_of(g0 * (2 * CK), 8)
            pltpu.async_copy(sd_h.at[pl.ds(off, 2 * CK)], sd_b, sdsem)

        def gstart(sd_b, dst_b, gs_b, gd_b, sdsem, gsem):
            pltpu.make_async_copy(
                sd_h.at[pl.ds(0, 2 * CK)], sd_b, sdsem).wait()
            for g in range(CK // 16):
                dst_b[pl.ds(g * 16, 16)] = sd_b[pl.ds(CK + g * 16, 16)]
            pltpu.async_copy(aux_h.at[sd_b.at[pl.ds(0, CK)]], gs_b, gsem)
            pltpu.async_copy(aux_h.at[dst_b], gd_b, gsem)

        def proc(nxt, sd_b, dst_b, gs_b, gd_b, sdsem, gsem):
            pltpu.make_async_copy(aux_h.at[dst_b], gs_b, gsem).wait()
            pltpu.make_async_copy(aux_h.at[dst_b], gd_b, gsem).wait()
            for g in range(CK // 16):
                rows = iota16 + g * 16
                d16 = dst_b[pl.ds(g * 16, 16)]
                for h in range(hpd):
                    if shared_aux:
                        cs, cd = zero16i, zero16i + 1
                    else:
                        cs = zero16i + (cidx * hpd + h)
                        cd = zero16i + (NC * hpd + cidx * hpd + h)
                    a = (plsc.load_gather(gs_b, [rows, cs])
                         + plsc.load_gather(gd_b, [rows, cd]))
                    a = jnp.where(a > 0, a, 0.2 * a)
                    plsc.addupdate_scatter(
                        den_l, [d16 + h * N], jnp.exp(a))
            fetch(nxt, sd_b, sdsem)

        fetch(0, sdA, sdsemA)
        fetch(1, sdB, sdsemB)
        gstart(sdA, dstA, gsA, gdA, sdsemA, semA)
        gstart(sdB, dstB, gsB, gdB, sdsemB, semB)

        def pair(i, _):
            proc(2 * i + 2, sdA, dstA, gsA, gdA, sdsemA, semA)
            gstart(sdA, dstA, gsA, gdA, sdsemA, semA)
            proc(2 * i + 3, sdB, dstB, gsB, gdB, sdsemB, semB)
            gstart(sdB, dstB, gsB, gdB, sdsemB, semB)
            return 0
        lax.fori_loop(0, NPAIR, pair, 0)
        proc(NCHUNK, sdA, dstA, gsA, gdA, sdsemA, semA)
        pltpu.make_async_copy(aux_h.at[dstB], gsB, semB).wait()
        pltpu.make_async_copy(aux_h.at[dstB], gdB, semB).wait()
        pltpu.make_async_copy(sd_h.at[pl.ds(0, 2 * CK)], sdA, sdsemA).wait()
        pltpu.sync_copy(den_l, out_h.at[cidx, sidx, 0])

    return k(aux128, sd)


def _reduce16_body(x_ref, o_ref):
    o_ref[...] = jnp.sum(x_ref[...], axis=0, keepdims=True)


def _reduce16(x):
    n, m = x.shape
    return pl.pallas_call(
        _reduce16_body,
        grid=(1,),
        in_specs=[pl.BlockSpec((n, m), lambda i: (0, 0))],
        out_specs=pl.BlockSpec((1, m), lambda i: (0, 0)),
        out_shape=jax.ShapeDtypeStruct((1, m), jnp.float32),
    )(x)


def _edge_gat(hv, auxt3, sd, bias, den, nseg, hpc, shared_aux):
    ndc = den.shape[0]
    mesh = plsc.VectorSubcoreMesh(core_axis_name="c", subcore_axis_name="s",
                                  num_cores=NC, num_subcores=NS)

    @functools.partial(
        pl.kernel, mesh=mesh,
        compiler_params=pltpu.CompilerParams(needs_layout_passes=False),
        out_type=jax.ShapeDtypeStruct((N, nseg * 128), jnp.float32),
        scratch_types=[
            pltpu.VMEM_SHARED((N, 128), jnp.float32),
            pltpu.VMEM((N,), jnp.float32),
            pltpu.VMEM((N,), jnp.float32),
            pltpu.VMEM((NPT,), jnp.float32),
            pltpu.VMEM((128,), jnp.float32),
            pltpu.VMEM((2 * CK,), jnp.int32),
            pltpu.VMEM((CK,), jnp.int32),
            pltpu.VMEM((CK,), jnp.int32),
            pltpu.VMEM((CK,), jnp.float32),
            pltpu.VMEM((2 * CK,), jnp.int32),
            pltpu.VMEM((CK,), jnp.int32),
            pltpu.VMEM((CK,), jnp.int32),
            pltpu.VMEM((CK,), jnp.float32),
            pltpu.VMEM((CK, 128), jnp.float32),
            pltpu.VMEM((CK, 128), jnp.float32),
            pltpu.VMEM((FC,), jnp.float32),
            pltpu.VMEM((FC,), jnp.float32),
            pltpu.SemaphoreType.DMA,
            pltpu.SemaphoreType.DMA,
            pltpu.SemaphoreType.DMA,
            pltpu.SemaphoreType.DMA,
            pltpu.SemaphoreType.DMA,
            pltpu.SemaphoreType.DMA,
        ],
    )
    def k(hv_h, auxt_h, sd_h, bias_h, den_h, out_h,
          acc, asrc_v, adst_v, den_v, bias_v,
          sdA, dstA, idxA, wbufA, sdB, dstB, idxB, wbufB,
          gbufA, gbufB, wsbuf, invbuf,
          gsemA, gsemB, ssemA, ssemB, sdsemA, sdsemB):
        cidx = lax.axis_index("c")
        sidx = lax.axis_index("s")
        zero16 = jnp.zeros((16,), jnp.float32)
        iota16 = lax.iota(jnp.int32, 16)
        nstart, ncht = _tile_rows(sidx)

        def head_body(hd, _):
            hdg = cidx * hpc + hd
            if shared_aux:
                a_row = jnp.int32(0)
                b_row = jnp.int32(1)
                d_row = jnp.int32(0)
            else:
                a_row = hdg
                b_row = nseg + hdg
                d_row = hdg
            pltpu.sync_copy(
                den_h.at[d_row, 0,
                         pl.ds(pl.multiple_of(nstart, 8), NPT)], den_v)
            pltpu.sync_copy(auxt_h.at[a_row, 0], asrc_v)
            pltpu.sync_copy(auxt_h.at[b_row, 0], adst_v)
            pltpu.sync_copy(bias_h.at[hdg, 0], bias_v)
            bj = [bias_v[pl.ds(j * 16, 16)] for j in range(8)]

            def zrow(r, _):
                for j in range(8):
                    gbufB[r, pl.ds(j * 16, 16)] = zero16
                return 0
            lax.fori_loop(0, FC, zrow, 0)

            def zchunk(i, _):
                r0 = pl.multiple_of(nstart + i * FC, 8)
                pltpu.sync_copy(gbufB, acc.at[pl.ds(r0, FC)])
                return 0
            lax.fori_loop(0, ncht, zchunk, 0)
            plsc.subcore_barrier()

            def fetch(ck, sd_b, sdsem):
                g0 = jnp.minimum(sidx * NCHUNK + ck, NS * NCHUNK - 1)
                off = pl.multiple_of(g0 * (2 * CK), 8)
                pltpu.async_copy(sd_h.at[pl.ds(off, 2 * CK)], sd_b, sdsem)

            def gstart(sd_b, dst_b, idx_b, gbuf_b, sdsem, gsem):
                pltpu.make_async_copy(
                    sd_h.at[pl.ds(0, 2 * CK)], sd_b, sdsem).wait()
                for g in range(CK // 16):
                    s16 = sd_b[pl.ds(g * 16, 16)]
                    idx_b[pl.ds(g * 16, 16)] = s16 * nseg + hdg
                    dst_b[pl.ds(g * 16, 16)] = sd_b[pl.ds(CK + g * 16, 16)]
                pltpu.async_copy(hv_h.at[idx_b], gbuf_b, gsem)

            def proc(nxt, sd_b, dst_b, idx_b, wbuf_b, gbuf_b, sdsem, sem):
                for g in range(CK // 16):
                    s16 = sd_b[pl.ds(g * 16, 16)]
                    d16 = dst_b[pl.ds(g * 16, 16)]
                    a = (plsc.load_gather(asrc_v, [s16])
                         + plsc.load_gather(adst_v, [d16]))
                    a = jnp.where(a > 0, a, 0.2 * a)
                    wbuf_b[pl.ds(g * 16, 16)] = jnp.exp(a)
                fetch(nxt, sd_b, sdsem)
                pltpu.make_async_copy(hv_h.at[idx_b], gbuf_b, sem).wait()

                def scale(g, _):
                    w16 = wbuf_b[pl.ds(g * 16, 16)]
                    for e in range(16):
                        w_e = w16[e]
                        row = g * 16 + e
                        for j in range(8):
                            sl = pl.ds(j * 16, 16)
                            gbuf_b[row, sl] = w_e * gbuf_b[row, sl]
                    return 0
                lax.fori_loop(0, CK // 16, scale, 0)

            fetch(0, sdA, sdsemA)
            fetch(1, sdB, sdsemB)
            gstart(sdA, dstA, idxA, gbufA, sdsemA, gsemA)
            gstart(sdB, dstB, idxB, gbufB, sdsemB, gsemB)

            def pair(i, _):
                proc(2 * i + 2, sdA, dstA, idxA, wbufA, gbufA, sdsemA, gsemA)
                pltpu.async_copy(gbufA, acc.at[dstA], ssemA, add=True)
                proc(2 * i + 3, sdB, dstB, idxB, wbufB, gbufB, sdsemB, gsemB)
                pltpu.async_copy(gbufB, acc.at[dstB], ssemB, add=True)
                pltpu.make_async_copy(gbufA, acc.at[dstA], ssemA).wait()
                gstart(sdA, dstA, idxA, gbufA, sdsemA, gsemA)
                pltpu.make_async_copy(gbufB, acc.at[dstB], ssemB).wait()
                gstart(sdB, dstB, idxB, gbufB, sdsemB, gsemB)
                return 0
            lax.fori_loop(0, NPAIR, pair, 0)
            proc(NCHUNK, sdA, dstA, idxA, wbufA, gbufA, sdsemA, gsemA)
            pltpu.sync_copy(gbufA, acc.at[dstA], add=True)
            pltpu.make_async_copy(hv_h.at[idxB], gbufB, gsemB).wait()
            pltpu.make_async_copy(
                sd_h.at[pl.ds(0, 2 * CK)], sdA, sdsemA).wait()
            plsc.subcore_barrier()

            def fchunk(i, _):
                n0 = pl.multiple_of(nstart + i * FC, 8)
                pltpu.sync_copy(acc.at[pl.ds(n0, FC)], gbufA)
                for g in range(FC // 16):
                    i16 = iota16 + (n0 + g * 16)
                    idxA[pl.ds(g * 16, 16)] = i16 * nseg + hdg
                cpf = pltpu.async_copy(hv_h.at[idxA], gbufB, gsemA)
                for g in range(FC // 16):
                    sl = pl.ds(g * 16, 16)
                    o16 = pl.multiple_of(n0 + g * 16, 8)
                    a = asrc_v[pl.ds(o16, 16)] + adst_v[pl.ds(o16, 16)]
                    a = jnp.where(a > 0, a, 0.2 * a)
                    ws = jnp.exp(a)
                    den16 = den_v[pl.ds(
                        pl.multiple_of(i * FC + g * 16, 8), 16)]
                    wsbuf[sl] = ws
                    invbuf[sl] = 1.0 / (den16 + ws + 1e-16)
                cpf.wait()

                def fnode(g, _):
                    ws16 = wsbuf[pl.ds(g * 16, 16)]
                    inv16 = invbuf[pl.ds(g * 16, 16)]
                    for e in range(16):
                        ws = ws16[e]
                        inv = inv16[e]
                        row = g * 16 + e
                        for j in range(8):
                            sl = pl.ds(j * 16, 16)
                            v = ((gbufA[row, sl] + ws * gbufB[row, sl]) * inv
                                 + bj[j])
                            gbufB[row, sl] = jnp.where(
                                v > 0, v, jnp.exp(v) - 1.0)
                    return 0
                lax.fori_loop(0, FC // 16, fnode, 0)
                c0 = pl.multiple_of(hdg * 128, 128)
                pltpu.sync_copy(gbufB,
                                out_h.at[pl.ds(n0, FC), pl.ds(c0, 128)])
                return 0
            lax.fori_loop(0, ncht, fchunk, 0)
            plsc.subcore_barrier()
            return 0

        lax.fori_loop(0, hpc, head_body, 0)

    return k(hv, auxt3, sd, bias, den)


def kernel(x, W1, a_src1, a_dst1, b1, W2, a_src2, a_dst2, b2,
           Wc1, bc1, Wc2, bc2, edge_index, batch_vec):
    src = edge_index[0].astype(jnp.int32)
    dst = edge_index[1].astype(jnp.int32)
    sd = jnp.concatenate(
        [src.reshape(E // CK, CK), dst.reshape(E // CK, CK)],
        axis=1).reshape(-1)

    eye1 = jnp.eye(HEADS, dtype=jnp.float32)
    bd_src1 = (a_src1[0][:, :, None] * eye1[:, None, :]).reshape(
        HEADS * HID, HEADS)
    bd_dst1 = (a_dst1[0][:, :, None] * eye1[:, None, :]).reshape(
        HEADS * HID, HEADS)
    amat1 = jnp.concatenate(
        [bd_src1, bd_dst1,
         jnp.zeros((HEADS * HID, 112), jnp.float32)], axis=1)
    amat2 = jnp.concatenate(
        [a_src2[0].T, a_dst2[0].T,
         jnp.zeros((OUT_CH, 126), jnp.float32)], axis=1)


    H1, AUX1 = _mm_aux(x, W1.T, amat1, bn=1000)
    denp1 = _den_pass(AUX1, sd, hpd=HEADS // NC, shared_aux=False)
    d1 = jnp.concatenate(
        [_reduce16(denp1[0].reshape(NS, 4 * N)),
         _reduce16(denp1[1].reshape(NS, 4 * N))], axis=0)
    den1 = jnp.pad(d1.reshape(HEADS, N),
                   ((0, 0), (0, NPAD - N))).reshape(HEADS, 1, NPAD)
    auxt31 = AUX1[:, :2 * HEADS].T.reshape(2 * HEADS, 1, N)
    h1 = _edge_gat(H1.reshape(N * HEADS, HID), auxt31, sd,
                   b1.reshape(HEADS, 1, HID), den1,
                   nseg=HEADS, hpc=HEADS // NC, shared_aux=False)

    H2, AUX2 = _mm_aux(h1, W2.T, amat2, bn=1000)
    denp2 = _den_pass(AUX2, sd, hpd=1, shared_aux=True)
    d2 = _reduce16(denp2[0].reshape(NS, N))
    den2 = jnp.pad(d2, ((0, 0), (0, NPAD - N))).reshape(1, 1, NPAD)
    auxt32 = AUX2[:, :2].T.reshape(2, 1, N)
    h2 = _edge_gat(H2.reshape(N * 2, 128), auxt32, sd,
                   b2.reshape(2, 1, 128), den2,
                   nseg=2, hpc=1, shared_aux=True)

    bv = batch_vec.astype(jnp.int32).reshape(N, 1)
    wc2tp = jnp.concatenate(
        [Wc2.T, jnp.zeros((OUT_CH, 128 - N_CLASSES), jnp.float32)], axis=1)
    bc2p = jnp.concatenate(
        [bc2, jnp.zeros((128 - N_CLASSES,), jnp.float32)]).reshape(1, 128)
    logits_p = _pool_classify(h2, bv, Wc1.T, bc1.reshape(1, OUT_CH),
                              wc2tp, bc2p, bn=1000)
    return logits_p[:, :N_CLASSES]


# speedup vs baseline: 24.7712x; 1.1951x over previous
import functools
import jax
import jax.numpy as jnp
from jax import lax
from jax.experimental import pallas as pl
from jax.experimental.pallas import tpu as pltpu
from jax.experimental.pallas import tpu_sc as plsc

N = 10000
E = 160000
N_GRAPHS = 200
IN_CH = 256
HID = 128
HEADS = 8
OUT_CH = 256
N_CLASSES = 14

NC = 2
NS = 16
CK = 80
EPT = E // NS
NCHUNK = EPT // CK
NPAIR = (NCHUNK - 1) // 2
NPT = 640
FC = 80
NPAD = NS * NPT


def _tile_rows(sidx):
    nstart = sidx * NPT
    ncht = jnp.where(sidx < NS - 1, NPT // FC, (N - NPT * (NS - 1)) // FC)
    return nstart, ncht


def _mm_aux_body(x_ref, w_ref, a_ref, h_ref, aux_ref):
    h = jnp.dot(x_ref[...], w_ref[...], preferred_element_type=jnp.float32)
    h_ref[...] = h
    aux_ref[...] = jnp.dot(h, a_ref[...], preferred_element_type=jnp.float32)


def _mm_aux(x, wt, amat, bn):
    n, k = x.shape
    m = wt.shape[1]
    aw = amat.shape[1]
    return pl.pallas_call(
        _mm_aux_body,
        grid=(n // bn,),
        in_specs=[
            pl.BlockSpec((bn, k), lambda i: (i, 0)),
            pl.BlockSpec((k, m), lambda i: (0, 0)),
            pl.BlockSpec((m, aw), lambda i: (0, 0)),
        ],
        out_specs=[
            pl.BlockSpec((bn, m), lambda i: (i, 0)),
            pl.BlockSpec((bn, aw), lambda i: (i, 0)),
        ],
        out_shape=[
            jax.ShapeDtypeStruct((n, m), jnp.float32),
            jax.ShapeDtypeStruct((n, aw), jnp.float32),
        ],
    )(x, wt, amat)


def _pool_body(h2_ref, bv_ref, wc1_ref, bc1_ref, wc2_ref, bc2_ref,
               out_ref, sum_acc, cnt_acc):
    i = pl.program_id(0)

    @pl.when(i == 0)
    def _():
        sum_acc[...] = jnp.zeros_like(sum_acc)
        cnt_acc[...] = jnp.zeros_like(cnt_acc)

    bv = bv_ref[...]
    oh = (bv == lax.broadcasted_iota(jnp.int32, (1, N_GRAPHS), 1)
          ).astype(jnp.float32)
    h2 = h2_ref[...]
    dn = (((0,), (0,)), ((), ()))
    sum_acc[...] += lax.dot_general(oh, h2, dn,
                                    preferred_element_type=jnp.float32)
    cnt_acc[...] += lax.dot_general(oh, jnp.ones_like(h2), dn,
                                    preferred_element_type=jnp.float32)

    @pl.when(i == pl.num_programs(0) - 1)
    def _():
        pooled = sum_acc[...] / jnp.maximum(cnt_acc[...], 1.0)
        z = jnp.maximum(
            jnp.dot(pooled, wc1_ref[...], preferred_element_type=jnp.float32)
            + bc1_ref[...], 0.0)
        out_ref[...] = jnp.dot(
            z, wc2_ref[...], preferred_element_type=jnp.float32) + bc2_ref[...]


def _pool_classify(h2, bv, wc1t, bc1, wc2tp, bc2p, bn):
    n, d = h2.shape
    return pl.pallas_call(
        _pool_body,
        grid=(n // bn,),
        in_specs=[
            pl.BlockSpec((bn, d), lambda i: (i, 0)),
            pl.BlockSpec((bn, 1), lambda i: (i, 0)),
            pl.BlockSpec(wc1t.shape, lambda i: (0, 0)),
            pl.BlockSpec(bc1.shape, lambda i: (0, 0)),
            pl.BlockSpec(wc2tp.shape, lambda i: (0, 0)),
            pl.BlockSpec(bc2p.shape, lambda i: (0, 0)),
        ],
        out_specs=pl.BlockSpec((N_GRAPHS, 128), lambda i: (0, 0)),
        out_shape=jax.ShapeDtypeStruct((N_GRAPHS, 128), jnp.float32),
        scratch_shapes=[
            pltpu.VMEM((N_GRAPHS, d), jnp.float32),
            pltpu.VMEM((N_GRAPHS, d), jnp.float32),
        ],
    )(h2, bv, wc1t, bc1, wc2tp, bc2p)


def _den_pass(auxt3, sd, hpd, shared_aux):
    mesh = plsc.VectorSubcoreMesh(core_axis_name="c", subcore_axis_name="s",
                                  num_cores=NC, num_subcores=NS)

    @functools.partial(
        pl.kernel, mesh=mesh,
        compiler_params=pltpu.CompilerParams(needs_layout_passes=False),
        out_type=jax.ShapeDtypeStruct((NC, NS, 1, hpd * N), jnp.float32),
        scratch_types=[
            pltpu.VMEM((hpd * N,), jnp.float32),
            pltpu.VMEM((hpd * N,), jnp.float32),
            pltpu.VMEM((hpd * N,), jnp.float32),
            pltpu.VMEM((2 * CK,), jnp.int32),
            pltpu.VMEM((2 * CK,), jnp.int32),
            pltpu.SemaphoreType.DMA,
            pltpu.SemaphoreType.DMA,
        ],
    )
    def k(auxt_h, sd_h, out_h,
          den_l, asr, adr, sdA, sdB, sdsemA, sdsemB):
        cidx = lax.axis_index("c")
        sidx = lax.axis_index("s")
        zero16 = jnp.zeros((16,), jnp.float32)

        for h in range(hpd):
            if shared_aux:
                ar, br = 0, 1
            else:
                ar, br = cidx * hpd + h, NC * hpd + cidx * hpd + h
            pltpu.sync_copy(auxt_h.at[ar, 0], asr.at[pl.ds(h * N, N)])
            pltpu.sync_copy(auxt_h.at[br, 0], adr.at[pl.ds(h * N, N)])

        def zloop(r, _):
            den_l[pl.ds(r * 16, 16)] = zero16
            return 0
        lax.fori_loop(0, hpd * N // 16, zloop, 0)

        def fetch(ck, sd_b, sdsem):
            g0 = jnp.minimum(sidx * NCHUNK + ck, NS * NCHUNK - 1)
            off = pl.multiple_of(g0 * (2 * CK), 8)
            pltpu.async_copy(sd_h.at[pl.ds(off, 2 * CK)], sd_b, sdsem)

        def proc(nxt, sd_b, sdsem):
            pltpu.make_async_copy(
                sd_h.at[pl.ds(0, 2 * CK)], sd_b, sdsem).wait()
            for g in range(CK // 16):
                s16 = sd_b[pl.ds(g * 16, 16)]
                d16 = sd_b[pl.ds(CK + g * 16, 16)]
                for h in range(hpd):
                    a = (plsc.load_gather(asr, [s16 + h * N])
                         + plsc.load_gather(adr, [d16 + h * N]))
                    a = jnp.where(a > 0, a, 0.2 * a)
                    plsc.addupdate_scatter(
                        den_l, [d16 + h * N], jnp.exp(a))
            fetch(nxt, sd_b, sdsem)

        fetch(0, sdA, sdsemA)
        fetch(1, sdB, sdsemB)

        def pair(i, _):
            proc(2 * i + 2, sdA, sdsemA)
            proc(2 * i + 3, sdB, sdsemB)
            return 0
        lax.fori_loop(0, NPAIR, pair, 0)
        proc(NCHUNK, sdA, sdsemA)
        pltpu.make_async_copy(sd_h.at[pl.ds(0, 2 * CK)], sdA, sdsemA).wait()
        pltpu.make_async_copy(sd_h.at[pl.ds(0, 2 * CK)], sdB, sdsemB).wait()
        pltpu.sync_copy(den_l, out_h.at[cidx, sidx, 0])

    return k(auxt3, sd)


def _reduce16_body(x_ref, o_ref):
    o_ref[...] = jnp.sum(x_ref[...], axis=0, keepdims=True)


def _reduce16(x):
    n, m = x.shape
    return pl.pallas_call(
        _reduce16_body,
        grid=(1,),
        in_specs=[pl.BlockSpec((n, m), lambda i: (0, 0))],
        out_specs=pl.BlockSpec((1, m), lambda i: (0, 0)),
        out_shape=jax.ShapeDtypeStruct((1, m), jnp.float32),
    )(x)


def _edge_gat(hv, auxt3, sd, bias, den, nseg, hpc, shared_aux):
    ndc = den.shape[0]
    mesh = plsc.VectorSubcoreMesh(core_axis_name="c", subcore_axis_name="s",
                                  num_cores=NC, num_subcores=NS)

    @functools.partial(
        pl.kernel, mesh=mesh,
        compiler_params=pltpu.CompilerParams(needs_layout_passes=False),
        out_type=jax.ShapeDtypeStruct((N, nseg * 128), jnp.float32),
        scratch_types=[
            pltpu.VMEM_SHARED((N, 128), jnp.float32),
            pltpu.VMEM((N,), jnp.float32),
            pltpu.VMEM((N,), jnp.float32),
            pltpu.VMEM((NPT,), jnp.float32),
            pltpu.VMEM((128,), jnp.float32),
            pltpu.VMEM((2 * CK,), jnp.int32),
            pltpu.VMEM((CK,), jnp.int32),
            pltpu.VMEM((CK,), jnp.int32),
            pltpu.VMEM((CK,), jnp.float32),
            pltpu.VMEM((2 * CK,), jnp.int32),
            pltpu.VMEM((CK,), jnp.int32),
            pltpu.VMEM((CK,), jnp.int32),
            pltpu.VMEM((CK,), jnp.float32),
            pltpu.VMEM((CK, 128), jnp.float32),
            pltpu.VMEM((CK, 128), jnp.float32),
            pltpu.VMEM((FC,), jnp.float32),
            pltpu.VMEM((FC,), jnp.float32),
            pltpu.SemaphoreType.DMA,
            pltpu.SemaphoreType.DMA,
            pltpu.SemaphoreType.DMA,
            pltpu.SemaphoreType.DMA,
            pltpu.SemaphoreType.DMA,
            pltpu.SemaphoreType.DMA,
        ],
    )
    def k(hv_h, auxt_h, sd_h, bias_h, den_h, out_h,
          acc, asrc_v, adst_v, den_v, bias_v,
          sdA, dstA, idxA, wbufA, sdB, dstB, idxB, wbufB,
          gbufA, gbufB, wsbuf, invbuf,
          gsemA, gsemB, ssemA, ssemB, sdsemA, sdsemB):
        cidx = lax.axis_index("c")
        sidx = lax.axis_index("s")
        zero16 = jnp.zeros((16,), jnp.float32)
        iota16 = lax.iota(jnp.int32, 16)
        nstart, ncht = _tile_rows(sidx)

        def head_body(hd, _):
            hdg = cidx * hpc + hd
            if shared_aux:
                a_row = jnp.int32(0)
                b_row = jnp.int32(1)
                d_row = jnp.int32(0)
            else:
                a_row = hdg
                b_row = nseg + hdg
                d_row = hdg
            pltpu.sync_copy(
                den_h.at[d_row, 0,
                         pl.ds(pl.multiple_of(nstart, 8), NPT)], den_v)
            pltpu.sync_copy(auxt_h.at[a_row, 0], asrc_v)
            pltpu.sync_copy(auxt_h.at[b_row, 0], adst_v)
            pltpu.sync_copy(bias_h.at[hdg, 0], bias_v)
            bj = [bias_v[pl.ds(j * 16, 16)] for j in range(8)]

            def zrow(r, _):
                for j in range(8):
                    gbufB[r, pl.ds(j * 16, 16)] = zero16
                return 0
            lax.fori_loop(0, FC, zrow, 0)

            def zchunk(i, _):
                r0 = pl.multiple_of(nstart + i * FC, 8)
                pltpu.sync_copy(gbufB, acc.at[pl.ds(r0, FC)])
                return 0
            lax.fori_loop(0, ncht, zchunk, 0)
            plsc.subcore_barrier()

            def fetch(ck, sd_b, sdsem):
                g0 = jnp.minimum(sidx * NCHUNK + ck, NS * NCHUNK - 1)
                off = pl.multiple_of(g0 * (2 * CK), 8)
                pltpu.async_copy(sd_h.at[pl.ds(off, 2 * CK)], sd_b, sdsem)

            def gstart(sd_b, dst_b, idx_b, gbuf_b, sdsem, gsem):
                pltpu.make_async_copy(
                    sd_h.at[pl.ds(0, 2 * CK)], sd_b, sdsem).wait()
                for g in range(CK // 16):
                    s16 = sd_b[pl.ds(g * 16, 16)]
                    idx_b[pl.ds(g * 16, 16)] = s16 * nseg + hdg
                    dst_b[pl.ds(g * 16, 16)] = sd_b[pl.ds(CK + g * 16, 16)]
                pltpu.async_copy(hv_h.at[idx_b], gbuf_b, gsem)

            def proc(nxt, sd_b, dst_b, idx_b, wbuf_b, gbuf_b, sdsem, sem):
                for g in range(CK // 16):
                    s16 = sd_b[pl.ds(g * 16, 16)]
                    d16 = dst_b[pl.ds(g * 16, 16)]
                    a = (plsc.load_gather(asrc_v, [s16])
                         + plsc.load_gather(adst_v, [d16]))
                    a = jnp.where(a > 0, a, 0.2 * a)
                    wbuf_b[pl.ds(g * 16, 16)] = jnp.exp(a)
                fetch(nxt, sd_b, sdsem)
                pltpu.make_async_copy(hv_h.at[idx_b], gbuf_b, sem).wait()

                def scale(g, _):
                    w16 = wbuf_b[pl.ds(g * 16, 16)]
                    for e in range(16):
                        w_e = w16[e]
                        row = g * 16 + e
                        for j in range(8):
                            sl = pl.ds(j * 16, 16)
                            gbuf_b[row, sl] = w_e * gbuf_b[row, sl]
                    return 0
                lax.fori_loop(0, CK // 16, scale, 0)

            fetch(0, sdA, sdsemA)
            fetch(1, sdB, sdsemB)
            gstart(sdA, dstA, idxA, gbufA, sdsemA, gsemA)
            gstart(sdB, dstB, idxB, gbufB, sdsemB, gsemB)

            def pair(i, _):
                proc(2 * i + 2, sdA, dstA, idxA, wbufA, gbufA, sdsemA, gsemA)
                pltpu.async_copy(gbufA, acc.at[dstA], ssemA, add=True)
                proc(2 * i + 3, sdB, dstB, idxB, wbufB, gbufB, sdsemB, gsemB)
                pltpu.async_copy(gbufB, acc.at[dstB], ssemB, add=True)
                pltpu.make_async_copy(gbufA, acc.at[dstA], ssemA).wait()
                gstart(sdA, dstA, idxA, gbufA, sdsemA, gsemA)
                pltpu.make_async_copy(gbufB, acc.at[dstB], ssemB).wait()
                gstart(sdB, dstB, idxB, gbufB, sdsemB, gsemB)
                return 0
            lax.fori_loop(0, NPAIR, pair, 0)
            proc(NCHUNK, sdA, dstA, idxA, wbufA, gbufA, sdsemA, gsemA)
            pltpu.sync_copy(gbufA, acc.at[dstA], add=True)
            pltpu.make_async_copy(hv_h.at[idxB], gbufB, gsemB).wait()
            pltpu.make_async_copy(
                sd_h.at[pl.ds(0, 2 * CK)], sdA, sdsemA).wait()
            plsc.subcore_barrier()

            def fchunk(i, _):
                n0 = pl.multiple_of(nstart + i * FC, 8)
                pltpu.sync_copy(acc.at[pl.ds(n0, FC)], gbufA)
                for g in range(FC // 16):
                    i16 = iota16 + (n0 + g * 16)
                    idxA[pl.ds(g * 16, 16)] = i16 * nseg + hdg
                cpf = pltpu.async_copy(hv_h.at[idxA], gbufB, gsemA)
                for g in range(FC // 16):
                    sl = pl.ds(g * 16, 16)
                    o16 = pl.multiple_of(n0 + g * 16, 8)
                    a = asrc_v[pl.ds(o16, 16)] + adst_v[pl.ds(o16, 16)]
                    a = jnp.where(a > 0, a, 0.2 * a)
                    ws = jnp.exp(a)
                    den16 = den_v[pl.ds(
                        pl.multiple_of(i * FC + g * 16, 8), 16)]
                    wsbuf[sl] = ws
                    invbuf[sl] = 1.0 / (den16 + ws + 1e-16)
                cpf.wait()

                def fnode(g, _):
                    ws16 = wsbuf[pl.ds(g * 16, 16)]
                    inv16 = invbuf[pl.ds(g * 16, 16)]
                    for e in range(16):
                        ws = ws16[e]
                        inv = inv16[e]
                        row = g * 16 + e
                        for j in range(8):
                            sl = pl.ds(j * 16, 16)
                            v = ((gbufA[row, sl] + ws * gbufB[row, sl]) * inv
                                 + bj[j])
                            gbufB[row, sl] = jnp.where(
                                v > 0, v, jnp.exp(v) - 1.0)
                    return 0
                lax.fori_loop(0, FC // 16, fnode, 0)
                c0 = pl.multiple_of(hdg * 128, 128)
                pltpu.sync_copy(gbufB,
                                out_h.at[pl.ds(n0, FC), pl.ds(c0, 128)])
                return 0
            lax.fori_loop(0, ncht, fchunk, 0)
            plsc.subcore_barrier()
            return 0

        lax.fori_loop(0, hpc, head_body, 0)

    return k(hv, auxt3, sd, bias, den)


def kernel(x, W1, a_src1, a_dst1, b1, W2, a_src2, a_dst2, b2,
           Wc1, bc1, Wc2, bc2, edge_index, batch_vec):
    src = edge_index[0].astype(jnp.int32)
    dst = edge_index[1].astype(jnp.int32)
    sd = jnp.concatenate(
        [src.reshape(E // CK, CK), dst.reshape(E // CK, CK)],
        axis=1).reshape(-1)

    eye1 = jnp.eye(HEADS, dtype=jnp.float32)
    bd_src1 = (a_src1[0][:, :, None] * eye1[:, None, :]).reshape(
        HEADS * HID, HEADS)
    bd_dst1 = (a_dst1[0][:, :, None] * eye1[:, None, :]).reshape(
        HEADS * HID, HEADS)
    amat1 = jnp.concatenate([bd_src1, bd_dst1], axis=1)
    amat2 = jnp.concatenate(
        [a_src2[0].T, a_dst2[0].T,
         jnp.zeros((OUT_CH, 14), jnp.float32)], axis=1)


    H1, AUX1 = _mm_aux(x, W1.T, amat1, bn=1000)
    auxt31 = AUX1.T.reshape(2 * HEADS, 1, N)
    denp1 = _den_pass(auxt31, sd, hpd=HEADS // NC, shared_aux=False)
    d1 = jnp.concatenate(
        [_reduce16(denp1[0].reshape(NS, 4 * N)),
         _reduce16(denp1[1].reshape(NS, 4 * N))], axis=0)
    den1 = jnp.pad(d1.reshape(HEADS, N),
                   ((0, 0), (0, NPAD - N))).reshape(HEADS, 1, NPAD)
    h1 = _edge_gat(H1.reshape(N * HEADS, HID), auxt31, sd,
                   b1.reshape(HEADS, 1, HID), den1,
                   nseg=HEADS, hpc=HEADS // NC, shared_aux=False)

    H2, AUX2 = _mm_aux(h1, W2.T, amat2, bn=1000)
    auxt32 = AUX2[:, :2].T.reshape(2, 1, N)
    denp2 = _den_pass(auxt32, sd, hpd=1, shared_aux=True)
    d2 = _reduce16(denp2[0].reshape(NS, N))
    den2 = jnp.pad(d2, ((0, 0), (0, NPAD - N))).reshape(1, 1, NPAD)
    h2 = _edge_gat(H2.reshape(N * 2, 128), auxt32, sd,
                   b2.reshape(2, 1, 128), den2,
                   nseg=2, hpc=1, shared_aux=True)

    bv = batch_vec.astype(jnp.int32).reshape(N, 1)
    wc2tp = jnp.concatenate(
        [Wc2.T, jnp.zeros((OUT_CH, 128 - N_CLASSES), jnp.float32)], axis=1)
    bc2p = jnp.concatenate(
        [bc2, jnp.zeros((128 - N_CLASSES,), jnp.float32)]).reshape(1, 128)
    logits_p = _pool_classify(h2, bv, Wc1.T, bc1.reshape(1, OUT_CH),
                              wc2tp, bc2p, bn=1000)
    return logits_p[:, :N_CLASSES]
